# baseline (speedup 1.0000x reference)
import functools

import jax
import jax.numpy as jnp
from jax import lax
from jax.experimental import pallas as pl
from jax.experimental.pallas import tpu as pltpu
from jax.experimental.pallas import tpu_sc as plsc

N = 6890
NP = 8192
NR = 64
F = 13776
FP = 13824
FPT = 864
HA = 778
HCPP = 2048
NCP = 512
SINK = 6900

INSIDE_W = 0.5
OUTSIDE_W = 0.005
CONTACT_W = 10.0
HCP_W = 1.0
POSE_W = 0.01
HPP_W = 0.01
ANGLE_W = 0.1
A1, A2 = 0.04, 0.04
B1, B2 = 0.07, 0.06
C1, C2 = 0.01, 0.01
D1, D2 = 0.023, 0.02
GEO_THRES = 0.3

_f32 = jnp.float32
_i32 = jnp.int32


def _zero2d(ref, nrows):
    def body(i, _):
        r = i // 8
        c = (i % 8) * 16
        ref[r, pl.ds(c, 16)] = jnp.zeros((16,), _f32)
        return 0
    lax.fori_loop(0, nrows * 8, body, 0)


def _cmask_body(ivc_h, cm_h, ivcl, acc):
    tid = lax.axis_index("s")

    @pl.when(tid == 0)
    def _():
        pltpu.sync_copy(ivc_h, ivcl)
        _zero2d(acc, NR)

        def cbody(k, _):
            ii = ivcl[pl.ds(k * 16, 16)]
            r = lax.shift_right_logical(ii, 7)
            c = lax.bitwise_and(ii, 127)
            plsc.store_scatter(acc, [r, c], jnp.ones((16,), _f32))
            return 0
        lax.fori_loop(0, NCP // 16, cbody, 0)
        pltpu.sync_copy(acc, cm_h)


def _cmask_call(ivc):
    mesh = plsc.VectorSubcoreMesh(core_axis_name="c", subcore_axis_name="s",
                                  num_cores=1)
    return pl.kernel(_cmask_body,
                     out_type=jax.ShapeDtypeStruct((NR, 128), _f32),
                     mesh=mesh,
                     scratch_types=[pltpu.VMEM((NCP,), _i32),
                                    pltpu.VMEM((NR, 128), _f32)],
                     compiler_params=pltpu.CompilerParams(
                         needs_layout_passes=False),
                     )(ivc)


def _normals_body(vx_h, vy_h, vz_h, f0_h, f1_h, f2_h,
                  vnx_h, vny_h, vnz_h,
                  vxl, vyl, vzl, f0l, f1l, f2l,
                  accx, accy, accz, rbuf,
                  svnx, svny, svnz):
    tid = lax.axis_index("s")
    pltpu.sync_copy(vx_h, vxl)
    pltpu.sync_copy(vy_h, vyl)
    pltpu.sync_copy(vz_h, vzl)
    pltpu.sync_copy(f0_h.at[pl.ds(tid * FPT, FPT)], f0l)
    pltpu.sync_copy(f1_h.at[pl.ds(tid * FPT, FPT)], f1l)
    pltpu.sync_copy(f2_h.at[pl.ds(tid * FPT, FPT)], f2l)

    _zero2d(accx, NR)
    _zero2d(accy, NR)
    _zero2d(accz, NR)

    def fbody(b, _):
        s = b * 16
        i0 = f0l[pl.ds(s, 16)]
        i1 = f1l[pl.ds(s, 16)]
        i2 = f2l[pl.ds(s, 16)]
        v0x = plsc.load_gather(vxl, [i0])
        v0y = plsc.load_gather(vyl, [i0])
        v0z = plsc.load_gather(vzl, [i0])
        v1x = plsc.load_gather(vxl, [i1])
        v1y = plsc.load_gather(vyl, [i1])
        v1z = plsc.load_gather(vzl, [i1])
        v2x = plsc.load_gather(vxl, [i2])
        v2y = plsc.load_gather(vyl, [i2])
        v2z = plsc.load_gather(vzl, [i2])
        e1x = v1x - v0x
        e1y = v1y - v0y
        e1z = v1z - v0z
        e2x = v2x - v0x
        e2y = v2y - v0y
        e2z = v2z - v0z
        fnx = e1y * e2z - e1z * e2y
        fny = e1z * e2x - e1x * e2z
        fnz = e1x * e2y - e1y * e2x
        for ii in (i0, i1, i2):
            r = lax.shift_right_logical(ii, 7)
            c = lax.bitwise_and(ii, 127)
            plsc.addupdate_scatter(accx, [r, c], fnx)
            plsc.addupdate_scatter(accy, [r, c], fny)
            plsc.addupdate_scatter(accz, [r, c], fnz)
        return 0
    lax.fori_loop(0, FPT // 16, fbody, 0)

    pltpu.sync_copy(accx, svnx.at[tid])
    pltpu.sync_copy(accy, svny.at[tid])
    pltpu.sync_copy(accz, svnz.at[tid])
    plsc.subcore_barrier()

    rpt = NR // 16
    for src, dst in ((svnx, vnx_h), (svny, vny_h), (svnz, vnz_h)):
        for s in range(16):
            pltpu.sync_copy(src.at[s, pl.ds(tid * rpt, rpt)], rbuf.at[s])

        def rbody(i, _):
            r = i // 8
            c = (i % 8) * 16
            tot = rbuf[0, r, pl.ds(c, 16)]
            for s in range(1, 16):
                tot = tot + rbuf[s, r, pl.ds(c, 16)]
            accx[r, pl.ds(c, 16)] = tot
            return 0
        lax.fori_loop(0, rpt * 8, rbody, 0)
        pltpu.sync_copy(accx.at[pl.ds(0, rpt)], dst.at[pl.ds(tid * rpt, rpt)])


def _normals_call(vx, vy, vz, f0, f1, f2):
    mesh = plsc.VectorSubcoreMesh(core_axis_name="c", subcore_axis_name="s",
                                  num_cores=1)
    out_type = [jax.ShapeDtypeStruct((NR, 128), _f32) for _ in range(3)]
    scratch = [
        pltpu.VMEM((NP,), _f32), pltpu.VMEM((NP,), _f32), pltpu.VMEM((NP,), _f32),
        pltpu.VMEM((FPT,), _i32), pltpu.VMEM((FPT,), _i32), pltpu.VMEM((FPT,), _i32),
        pltpu.VMEM((NR, 128), _f32), pltpu.VMEM((NR, 128), _f32),
        pltpu.VMEM((NR, 128), _f32),
        pltpu.VMEM((16, NR // 16, 128), _f32),
        pltpu.VMEM_SHARED((16, NR, 128), _f32),
        pltpu.VMEM_SHARED((16, NR, 128), _f32),
        pltpu.VMEM_SHARED((16, NR, 128), _f32),
    ]
    return pl.kernel(_normals_body, out_type=out_type, mesh=mesh,
                     scratch_types=scratch,
                     compiler_params=pltpu.CompilerParams(
                         needs_layout_passes=False),
                     )(vx, vy, vz, f0, f1, f2)


RB = 512
_GRID = 14


def _stream_body(a_ref, b_ref, sqr_ref, sqc_ref, g_ref, cm_ref,
                 min_ref, idx_ref, gc_ref):
    p = lax.dot_general(a_ref[...], b_ref[...], (((1,), (0,)), ((), ())),
                        preferred_element_type=_f32)
    d2 = sqr_ref[...] + sqc_ref[...] - 2.0 * p
    d2 = jnp.maximum(d2, 0.0)
    g = g_ref[...]
    d2m = jnp.where(g < GEO_THRES, 1e10, d2)
    m = jnp.min(d2m, axis=1)
    iota = lax.broadcasted_iota(_i32, d2m.shape, 1)
    idx = jnp.min(jnp.where(d2m == m[:, None], iota, N), axis=1)
    gc = jnp.min(jnp.where(cm_ref[...] > 0.5, g, 1e10), axis=1)
    min_ref[...] = m[:, None]
    idx_ref[...] = idx[:, None]
    gc_ref[...] = gc[:, None]


def _stream_call(a, b, sqr, sqc, geodist, cm):
    return pl.pallas_call(
        _stream_body,
        grid=(_GRID,),
        in_specs=[
            pl.BlockSpec((RB, 8), lambda i: (i, 0)),
            pl.BlockSpec((8, N), lambda i: (0, 0)),
            pl.BlockSpec((RB, 1), lambda i: (i, 0)),
            pl.BlockSpec((1, N), lambda i: (0, 0)),
            pl.BlockSpec((RB, N), lambda i: (i, 0)),
            pl.BlockSpec((1, N), lambda i: (0, 0)),
        ],
        out_specs=[
            pl.BlockSpec((RB, 1), lambda i: (i, 0)),
            pl.BlockSpec((RB, 1), lambda i: (i, 0)),
            pl.BlockSpec((RB, 1), lambda i: (i, 0)),
        ],
        out_shape=[
            jax.ShapeDtypeStruct((N, 1), _f32),
            jax.ShapeDtypeStruct((N, 1), _i32),
            jax.ShapeDtypeStruct((N, 1), _f32),
        ],
    )(a, b, sqr, sqc, geodist, cm)


_VPT = NP // 32
_HPT = HCPP // 32


def _gather_body(idx_h, md_h, vx_h, vy_h, vz_h, nx_h, ny_h, nz_h, hcp_h,
                 gxj_h, gyj_h, gzj_h, gnx_h, gny_h, gnz_h, hvi_h, vh_h,
                 idxl, mdl, vxl, vyl, vzl, nxl, nyl, nzl, hcpl,
                 ox, oy, oz, onx, ony, onz, ohv, ovh):
    nc = 2
    wid = lax.axis_index("s") * nc + lax.axis_index("c")
    pltpu.sync_copy(idx_h, idxl)
    pltpu.sync_copy(md_h, mdl)
    pltpu.sync_copy(vx_h, vxl)
    pltpu.sync_copy(vy_h, vyl)
    pltpu.sync_copy(vz_h, vzl)
    pltpu.sync_copy(nx_h, nxl)
    pltpu.sync_copy(ny_h, nyl)
    pltpu.sync_copy(nz_h, nzl)
    pltpu.sync_copy(hcp_h, hcpl)

    def vbody(b, _):
        s = wid * _VPT + b * 16
        jv = idxl[pl.ds(s, 16)]
        o = b * 16
        ox[pl.ds(o, 16)] = plsc.load_gather(vxl, [jv])
        oy[pl.ds(o, 16)] = plsc.load_gather(vyl, [jv])
        oz[pl.ds(o, 16)] = plsc.load_gather(vzl, [jv])
        onx[pl.ds(o, 16)] = plsc.load_gather(nxl, [jv])
        ony[pl.ds(o, 16)] = plsc.load_gather(nyl, [jv])
        onz[pl.ds(o, 16)] = plsc.load_gather(nzl, [jv])
        return 0
    lax.fori_loop(0, _VPT // 16, vbody, 0)

    def hbody(b, _):
        s = wid * _HPT + b * 16
        hh = hcpl[pl.ds(s, 16)]
        jh = plsc.load_gather(idxl, [hh])
        xhx = plsc.load_gather(vxl, [hh])
        xhy = plsc.load_gather(vyl, [hh])
        xhz = plsc.load_gather(vzl, [hh])
        xjx = plsc.load_gather(vxl, [jh])
        xjy = plsc.load_gather(vyl, [jh])
        xjz = plsc.load_gather(vzl, [jh])
        njx = plsc.load_gather(nxl, [jh])
        njy = plsc.load_gather(nyl, [jh])
        njz = plsc.load_gather(nzl, [jh])
        md = plsc.load_gather(mdl, [hh])
        ext = (njx * (xjx - xhx) + njy * (xjy - xhy) + njz * (xjz - xhz)) > 0.0
        isds = lax.bitwise_and(hh, 7) == 0
        hvi = jnp.where(jnp.logical_and(isds, jnp.logical_not(ext)),
                        jnp.ones((16,), _f32), jnp.zeros((16,), _f32))
        o = b * 16
        ohv[pl.ds(o, 16)] = hvi
        ovh[pl.ds(o, 16)] = md
        return 0
    lax.fori_loop(0, _HPT // 16, hbody, 0)

    pltpu.sync_copy(ox, gxj_h.at[pl.ds(wid * _VPT, _VPT)])
    pltpu.sync_copy(oy, gyj_h.at[pl.ds(wid * _VPT, _VPT)])
    pltpu.sync_copy(oz, gzj_h.at[pl.ds(wid * _VPT, _VPT)])
    pltpu.sync_copy(onx, gnx_h.at[pl.ds(wid * _VPT, _VPT)])
    pltpu.sync_copy(ony, gny_h.at[pl.ds(wid * _VPT, _VPT)])
    pltpu.sync_copy(onz, gnz_h.at[pl.ds(wid * _VPT, _VPT)])
    pltpu.sync_copy(ohv, hvi_h.at[pl.ds(wid * _HPT, _HPT)])
    pltpu.sync_copy(ovh, vh_h.at[pl.ds(wid * _HPT, _HPT)])


def _gather_call(idx_t, md_t, vx, vy, vz, nx, ny, nz, hcp):
    mesh = plsc.VectorSubcoreMesh(core_axis_name="c", subcore_axis_name="s",
                                  num_cores=2)
    out_type = [jax.ShapeDtypeStruct((NP,), _f32) for _ in range(6)] + \
               [jax.ShapeDtypeStruct((HCPP,), _f32),
                jax.ShapeDtypeStruct((HCPP,), _f32)]
    scratch = [
        pltpu.VMEM((NP,), _i32), pltpu.VMEM((NP,), _f32),
        pltpu.VMEM((NP,), _f32), pltpu.VMEM((NP,), _f32), pltpu.VMEM((NP,), _f32),
        pltpu.VMEM((NP,), _f32), pltpu.VMEM((NP,), _f32), pltpu.VMEM((NP,), _f32),
        pltpu.VMEM((HCPP,), _i32),
        pltpu.VMEM((_VPT,), _f32), pltpu.VMEM((_VPT,), _f32), pltpu.VMEM((_VPT,), _f32),
        pltpu.VMEM((_VPT,), _f32), pltpu.VMEM((_VPT,), _f32), pltpu.VMEM((_VPT,), _f32),
        pltpu.VMEM((_HPT,), _f32), pltpu.VMEM((_HPT,), _f32),
    ]
    return pl.kernel(_gather_body, out_type=out_type, mesh=mesh,
                     scratch_types=scratch,
                     compiler_params=pltpu.CompilerParams(
                         needs_layout_passes=False),
                     )(idx_t, md_t, vx, vy, vz, nx, ny, nz, hcp)


def _finish_body(md_ref, gc_ref, gxj_ref, gyj_ref, gzj_ref,
                 gnx_ref, gny_ref, gnz_ref, vnx_ref, vny_ref, vnz_ref,
                 xx_ref, xy_ref, xz_ref, ivx_ref, ivy_ref, ivz_ref,
                 hvi_ref, vh_ref, hw_ref, bp_ref, ip_ref, lhp_ref, rhp_ref,
                 out_ref):
    pos = (lax.broadcasted_iota(_i32, (NR, 128), 0) * 128
           + lax.broadcasted_iota(_i32, (NR, 128), 1))
    valid = pos < N
    v2v = jnp.sqrt(md_ref[...] + 1e-12)
    xx = xx_ref[...]
    xy = xy_ref[...]
    xz = xz_ref[...]
    gnx = gnx_ref[...]
    gny = gny_ref[...]
    gnz = gnz_ref[...]
    vnx = vnx_ref[...]
    vny = vny_ref[...]
    vnz = vnz_ref[...]
    ext = (gnx * (gxj_ref[...] - xx) + gny * (gyj_ref[...] - xy)
           + gnz * (gzj_ref[...] - xz)) > 0.0
    isds = jnp.logical_and((pos % 8) == 0, valid)
    inside = jnp.logical_and(isds, jnp.logical_not(ext))

    def mmean(v, m):
        c = jnp.sum(jnp.where(m, 1.0, 0.0))
        s = jnp.sum(jnp.where(m, v, 0.0))
        return jnp.where(c > 0, s / jnp.maximum(c, 1.0), 0.0)

    gc = gc_ref[...]
    wout = 1.0 / (5.0 * gc + 1.0)
    contact = CONTACT_W * mmean(A1 * wout * jnp.tanh(v2v / A2),
                                jnp.logical_and(isds, jnp.logical_not(inside)))
    insidel = INSIDE_W * mmean(B1 * jnp.tanh(v2v / B2), inside)
    ni = jnp.sqrt(vnx * vnx + vny * vny + vnz * vnz)
    nj = jnp.sqrt(gnx * gnx + gny * gny + gnz * gnz)
    ng = (vnx * gnx + vny * gny + vnz * gnz) / ((ni + 1e-8) * (nj + 1e-8))
    angle = ANGLE_W * mmean(1.0 + ng, jnp.logical_and(v2v < 0.01, valid))

    odel = jnp.sqrt((ivx_ref[...] - xx) ** 2 + (ivy_ref[...] - xy) ** 2
                    + (ivz_ref[...] - xz) ** 2 + 1e-12)
    outside = OUTSIDE_W * jnp.sum(jnp.where(valid, odel * (2.0 * gc) ** 2, 0.0))

    hpos = (lax.broadcasted_iota(_i32, (16, 128), 0) * 128
            + lax.broadcasted_iota(_i32, (16, 128), 1))
    lmask = hpos < HA
    rmask = jnp.logical_and(hpos >= HA, hpos < 2 * HA)
    hv = hvi_ref[...] > 0.5
    nhv = jnp.logical_not(hv)
    v2vh = jnp.sqrt(vh_ref[...] + 1e-12)
    w = -0.1 * hw_ref[...] + 1.0
    vout = w * (C1 * jnp.tanh(v2vh / C2))
    vin = D1 * jnp.tanh(v2vh / D2)
    hco = mmean(vout, jnp.logical_and(lmask, nhv)) + \
          mmean(vout, jnp.logical_and(rmask, nhv))
    hci = mmean(vin, jnp.logical_and(lmask, hv)) + \
          mmean(vin, jnp.logical_and(rmask, hv))
    hand = HCP_W * (hci + hco)

    bp = bp_ref[...] - ip_ref[...]
    pose = POSE_W * jnp.sum(bp * bp)
    hpp = HPP_W * (jnp.sum(lhp_ref[...] ** 2) + jnp.sum(rhp_ref[...] ** 2))
    total = contact + insidel + outside + angle + pose + hpp + hand
    out_ref[...] = total[None, None]


def _finish_call(*args):
    return pl.pallas_call(
        _finish_body,
        out_shape=jax.ShapeDtypeStruct((1, 1), _f32),
    )(*args)


@jax.jit
def kernel(vertices, body_pose, left_hand_pose, right_hand_pose, init_verts,
           init_pose, geodist, hand_contact_prior_weights, faces, ds,
           hand_contact_prior, init_verts_in_contact):
    x = vertices[0]
    vx = jnp.pad(x[:, 0], (0, NP - N))
    vy = jnp.pad(x[:, 1], (0, NP - N))
    vz = jnp.pad(x[:, 2], (0, NP - N))
    f0 = jnp.pad(faces[:, 0], (0, FP - F), constant_values=SINK)
    f1 = jnp.pad(faces[:, 1], (0, FP - F), constant_values=SINK)
    f2 = jnp.pad(faces[:, 2], (0, FP - F), constant_values=SINK)
    ivc = jnp.pad(init_verts_in_contact, (0, NCP - 400), constant_values=SINK)
    hcp = jnp.pad(hand_contact_prior, (0, HCPP - 2 * HA))

    cm2 = _cmask_call(ivc)
    vnx2, vny2, vnz2 = _normals_call(vx, vy, vz, f0, f1, f2)

    sq = jnp.sum(x * x, axis=1)
    zeros = jnp.zeros((NP,), _f32)
    a = jnp.stack([vx, vy, vz, zeros, zeros, zeros, zeros, zeros], 1)
    b = a.T[:, :N]
    cm = cm2.reshape(NP)[:N][None, :]
    mind2, idx, gc = _stream_call(a, b, sq.reshape(N, 1), sq[None, :],
                                  geodist, cm)

    idx_t = jnp.pad(idx[:, 0], (0, NP - N))
    md_t = jnp.pad(mind2[:, 0], (0, NP - N))
    gxj, gyj, gzj, gnx, gny, gnz, hvi, vh = _gather_call(
        idx_t, md_t, vx, vy, vz,
        vnx2.reshape(NP), vny2.reshape(NP), vnz2.reshape(NP), hcp)

    r2 = lambda v: v.reshape(NR, 128)
    gc_t = jnp.pad(gc[:, 0], (0, NP - N))
    ivx = jnp.pad(init_verts[0, :, 0], (0, NP - N))
    ivy = jnp.pad(init_verts[0, :, 1], (0, NP - N))
    ivz = jnp.pad(init_verts[0, :, 2], (0, NP - N))
    hw = jnp.pad(hand_contact_prior_weights, (0, HCPP - 2 * HA)).reshape(16, 128)
    bp = jnp.pad(body_pose[0], (0, 128 - 69))[None, :]
    ip = jnp.pad(init_pose[0], (0, 128 - 69))[None, :]
    lhp = jnp.pad(left_hand_pose[0], (0, 128 - 45))[None, :]
    rhp = jnp.pad(right_hand_pose[0], (0, 128 - 45))[None, :]

    out = _finish_call(r2(md_t), r2(gc_t), r2(gxj), r2(gyj), r2(gzj),
                       r2(gnx), r2(gny), r2(gnz), vnx2, vny2, vnz2,
                       r2(vx), r2(vy), r2(vz), r2(ivx), r2(ivy), r2(ivz),
                       hvi.reshape(16, 128), vh.reshape(16, 128), hw,
                       bp, ip, lhp, rhp)
    return out[0, 0]


# speedup vs baseline: 1.0007x; 1.0007x over previous
import functools

import jax
import jax.numpy as jnp
from jax import lax
from jax.experimental import pallas as pl
from jax.experimental.pallas import tpu as pltpu
from jax.experimental.pallas import tpu_sc as plsc

N = 6890
NP = 8192
NR = 64
F = 13776
FP = 13824
FPT = 864
HA = 778
HCPP = 2048
NCP = 512
SINK = 6900

INSIDE_W = 0.5
OUTSIDE_W = 0.005
CONTACT_W = 10.0
HCP_W = 1.0
POSE_W = 0.01
HPP_W = 0.01
ANGLE_W = 0.1
A1, A2 = 0.04, 0.04
B1, B2 = 0.07, 0.06
C1, C2 = 0.01, 0.01
D1, D2 = 0.023, 0.02
GEO_THRES = 0.3

_f32 = jnp.float32
_i32 = jnp.int32


def _zero2d(ref, nrows):
    def body(i, _):
        r = i // 8
        c = (i % 8) * 16
        ref[r, pl.ds(c, 16)] = jnp.zeros((16,), _f32)
        return 0
    lax.fori_loop(0, nrows * 8, body, 0)


def _cmask_body(ivc_h, cm_h, ivcl, acc):
    tid = lax.axis_index("s")

    @pl.when(tid == 0)
    def _():
        pltpu.sync_copy(ivc_h, ivcl)
        _zero2d(acc, NR)

        def cbody(k, _):
            ii = ivcl[pl.ds(k * 16, 16)]
            r = lax.shift_right_logical(ii, 7)
            c = lax.bitwise_and(ii, 127)
            plsc.store_scatter(acc, [r, c], jnp.ones((16,), _f32))
            return 0
        lax.fori_loop(0, NCP // 16, cbody, 0)
        pltpu.sync_copy(acc, cm_h)


def _cmask_call(ivc):
    mesh = plsc.VectorSubcoreMesh(core_axis_name="c", subcore_axis_name="s",
                                  num_cores=1)
    return pl.kernel(_cmask_body,
                     out_type=jax.ShapeDtypeStruct((NR, 128), _f32),
                     mesh=mesh,
                     scratch_types=[pltpu.VMEM((NCP,), _i32),
                                    pltpu.VMEM((NR, 128), _f32)],
                     compiler_params=pltpu.CompilerParams(
                         needs_layout_passes=False),
                     )(ivc)


def _normals_body(vx_h, vy_h, vz_h, f0_h, f1_h, f2_h,
                  vnx_h, vny_h, vnz_h,
                  vxl, vyl, vzl, f0l, f1l, f2l,
                  accx, accy, accz, rbuf,
                  svnx, svny, svnz):
    tid = lax.axis_index("s")
    pltpu.sync_copy(vx_h, vxl)
    pltpu.sync_copy(vy_h, vyl)
    pltpu.sync_copy(vz_h, vzl)
    pltpu.sync_copy(f0_h.at[pl.ds(tid * FPT, FPT)], f0l)
    pltpu.sync_copy(f1_h.at[pl.ds(tid * FPT, FPT)], f1l)
    pltpu.sync_copy(f2_h.at[pl.ds(tid * FPT, FPT)], f2l)

    _zero2d(accx, NR)
    _zero2d(accy, NR)
    _zero2d(accz, NR)

    def fbody(b, _):
        s = b * 16
        i0 = f0l[pl.ds(s, 16)]
        i1 = f1l[pl.ds(s, 16)]
        i2 = f2l[pl.ds(s, 16)]
        v0x = plsc.load_gather(vxl, [i0])
        v0y = plsc.load_gather(vyl, [i0])
        v0z = plsc.load_gather(vzl, [i0])
        v1x = plsc.load_gather(vxl, [i1])
        v1y = plsc.load_gather(vyl, [i1])
        v1z = plsc.load_gather(vzl, [i1])
        v2x = plsc.load_gather(vxl, [i2])
        v2y = plsc.load_gather(vyl, [i2])
        v2z = plsc.load_gather(vzl, [i2])
        e1x = v1x - v0x
        e1y = v1y - v0y
        e1z = v1z - v0z
        e2x = v2x - v0x
        e2y = v2y - v0y
        e2z = v2z - v0z
        fnx = e1y * e2z - e1z * e2y
        fny = e1z * e2x - e1x * e2z
        fnz = e1x * e2y - e1y * e2x
        for ii in (i0, i1, i2):
            r = lax.shift_right_logical(ii, 7)
            c = lax.bitwise_and(ii, 127)
            plsc.addupdate_scatter(accx, [r, c], fnx)
            plsc.addupdate_scatter(accy, [r, c], fny)
            plsc.addupdate_scatter(accz, [r, c], fnz)
        return 0
    lax.fori_loop(0, FPT // 16, fbody, 0)

    pltpu.sync_copy(accx, svnx.at[tid])
    pltpu.sync_copy(accy, svny.at[tid])
    pltpu.sync_copy(accz, svnz.at[tid])
    plsc.subcore_barrier()

    rpt = NR // 16
    for src, dst in ((svnx, vnx_h), (svny, vny_h), (svnz, vnz_h)):
        for s in range(16):
            pltpu.sync_copy(src.at[s, pl.ds(tid * rpt, rpt)], rbuf.at[s])

        def rbody(i, _):
            r = i // 8
            c = (i % 8) * 16
            tot = rbuf[0, r, pl.ds(c, 16)]
            for s in range(1, 16):
                tot = tot + rbuf[s, r, pl.ds(c, 16)]
            accx[r, pl.ds(c, 16)] = tot
            return 0
        lax.fori_loop(0, rpt * 8, rbody, 0)
        pltpu.sync_copy(accx.at[pl.ds(0, rpt)], dst.at[pl.ds(tid * rpt, rpt)])


def _normals_call(vx, vy, vz, f0, f1, f2):
    mesh = plsc.VectorSubcoreMesh(core_axis_name="c", subcore_axis_name="s",
                                  num_cores=1)
    out_type = [jax.ShapeDtypeStruct((NR, 128), _f32) for _ in range(3)]
    scratch = [
        pltpu.VMEM((NP,), _f32), pltpu.VMEM((NP,), _f32), pltpu.VMEM((NP,), _f32),
        pltpu.VMEM((FPT,), _i32), pltpu.VMEM((FPT,), _i32), pltpu.VMEM((FPT,), _i32),
        pltpu.VMEM((NR, 128), _f32), pltpu.VMEM((NR, 128), _f32),
        pltpu.VMEM((NR, 128), _f32),
        pltpu.VMEM((16, NR // 16, 128), _f32),
        pltpu.VMEM_SHARED((16, NR, 128), _f32),
        pltpu.VMEM_SHARED((16, NR, 128), _f32),
        pltpu.VMEM_SHARED((16, NR, 128), _f32),
    ]
    return pl.kernel(_normals_body, out_type=out_type, mesh=mesh,
                     scratch_types=scratch,
                     compiler_params=pltpu.CompilerParams(
                         needs_layout_passes=False),
                     )(vx, vy, vz, f0, f1, f2)


RB = 256
_GRID = 27


def _stream_body(a_ref, b_ref, sqr_ref, sqc_ref, g_ref, cm_ref,
                 min_ref, idx_ref, gc_ref):
    p = lax.dot_general(a_ref[...], b_ref[...], (((1,), (0,)), ((), ())),
                        preferred_element_type=_f32)
    d2 = sqr_ref[...] + sqc_ref[...] - 2.0 * p
    d2 = jnp.maximum(d2, 0.0)
    g = g_ref[...]
    d2m = jnp.where(g < GEO_THRES, 1e10, d2)
    m = jnp.min(d2m, axis=1)
    iota = lax.broadcasted_iota(_i32, d2m.shape, 1)
    idx = jnp.min(jnp.where(d2m == m[:, None], iota, N), axis=1)
    gc = jnp.min(jnp.where(cm_ref[...] > 0.5, g, 1e10), axis=1)
    min_ref[...] = m[:, None]
    idx_ref[...] = idx[:, None]
    gc_ref[...] = gc[:, None]


def _stream_call(a, b, sqr, sqc, geodist, cm):
    return pl.pallas_call(
        _stream_body,
        grid=(_GRID,),
        in_specs=[
            pl.BlockSpec((RB, 8), lambda i: (i, 0)),
            pl.BlockSpec((8, N), lambda i: (0, 0)),
            pl.BlockSpec((RB, 1), lambda i: (i, 0)),
            pl.BlockSpec((1, N), lambda i: (0, 0)),
            pl.BlockSpec((RB, N), lambda i: (i, 0)),
            pl.BlockSpec((1, N), lambda i: (0, 0)),
        ],
        out_specs=[
            pl.BlockSpec((RB, 1), lambda i: (i, 0)),
            pl.BlockSpec((RB, 1), lambda i: (i, 0)),
            pl.BlockSpec((RB, 1), lambda i: (i, 0)),
        ],
        out_shape=[
            jax.ShapeDtypeStruct((N, 1), _f32),
            jax.ShapeDtypeStruct((N, 1), _i32),
            jax.ShapeDtypeStruct((N, 1), _f32),
        ],
    )(a, b, sqr, sqc, geodist, cm)


_VPT = NP // 32
_HPT = HCPP // 32


def _gather_body(idx_h, md_h, vx_h, vy_h, vz_h, nx_h, ny_h, nz_h, hcp_h,
                 gxj_h, gyj_h, gzj_h, gnx_h, gny_h, gnz_h, hvi_h, vh_h,
                 idxl, mdl, vxl, vyl, vzl, nxl, nyl, nzl, hcpl,
                 ox, oy, oz, onx, ony, onz, ohv, ovh):
    nc = 2
    wid = lax.axis_index("s") * nc + lax.axis_index("c")
    pltpu.sync_copy(idx_h, idxl)
    pltpu.sync_copy(md_h, mdl)
    pltpu.sync_copy(vx_h, vxl)
    pltpu.sync_copy(vy_h, vyl)
    pltpu.sync_copy(vz_h, vzl)
    pltpu.sync_copy(nx_h, nxl)
    pltpu.sync_copy(ny_h, nyl)
    pltpu.sync_copy(nz_h, nzl)
    pltpu.sync_copy(hcp_h, hcpl)

    def vbody(b, _):
        s = wid * _VPT + b * 16
        jv = idxl[pl.ds(s, 16)]
        o = b * 16
        ox[pl.ds(o, 16)] = plsc.load_gather(vxl, [jv])
        oy[pl.ds(o, 16)] = plsc.load_gather(vyl, [jv])
        oz[pl.ds(o, 16)] = plsc.load_gather(vzl, [jv])
        onx[pl.ds(o, 16)] = plsc.load_gather(nxl, [jv])
        ony[pl.ds(o, 16)] = plsc.load_gather(nyl, [jv])
        onz[pl.ds(o, 16)] = plsc.load_gather(nzl, [jv])
        return 0
    lax.fori_loop(0, _VPT // 16, vbody, 0)

    def hbody(b, _):
        s = wid * _HPT + b * 16
        hh = hcpl[pl.ds(s, 16)]
        jh = plsc.load_gather(idxl, [hh])
        xhx = plsc.load_gather(vxl, [hh])
        xhy = plsc.load_gather(vyl, [hh])
        xhz = plsc.load_gather(vzl, [hh])
        xjx = plsc.load_gather(vxl, [jh])
        xjy = plsc.load_gather(vyl, [jh])
        xjz = plsc.load_gather(vzl, [jh])
        njx = plsc.load_gather(nxl, [jh])
        njy = plsc.load_gather(nyl, [jh])
        njz = plsc.load_gather(nzl, [jh])
        md = plsc.load_gather(mdl, [hh])
        ext = (njx * (xjx - xhx) + njy * (xjy - xhy) + njz * (xjz - xhz)) > 0.0
        isds = lax.bitwise_and(hh, 7) == 0
        hvi = jnp.where(jnp.logical_and(isds, jnp.logical_not(ext)),
                        jnp.ones((16,), _f32), jnp.zeros((16,), _f32))
        o = b * 16
        ohv[pl.ds(o, 16)] = hvi
        ovh[pl.ds(o, 16)] = md
        return 0
    lax.fori_loop(0, _HPT // 16, hbody, 0)

    pltpu.sync_copy(ox, gxj_h.at[pl.ds(wid * _VPT, _VPT)])
    pltpu.sync_copy(oy, gyj_h.at[pl.ds(wid * _VPT, _VPT)])
    pltpu.sync_copy(oz, gzj_h.at[pl.ds(wid * _VPT, _VPT)])
    pltpu.sync_copy(onx, gnx_h.at[pl.ds(wid * _VPT, _VPT)])
    pltpu.sync_copy(ony, gny_h.at[pl.ds(wid * _VPT, _VPT)])
    pltpu.sync_copy(onz, gnz_h.at[pl.ds(wid * _VPT, _VPT)])
    pltpu.sync_copy(ohv, hvi_h.at[pl.ds(wid * _HPT, _HPT)])
    pltpu.sync_copy(ovh, vh_h.at[pl.ds(wid * _HPT, _HPT)])


def _gather_call(idx_t, md_t, vx, vy, vz, nx, ny, nz, hcp):
    mesh = plsc.VectorSubcoreMesh(core_axis_name="c", subcore_axis_name="s",
                                  num_cores=2)
    out_type = [jax.ShapeDtypeStruct((NP,), _f32) for _ in range(6)] + \
               [jax.ShapeDtypeStruct((HCPP,), _f32),
                jax.ShapeDtypeStruct((HCPP,), _f32)]
    scratch = [
        pltpu.VMEM((NP,), _i32), pltpu.VMEM((NP,), _f32),
        pltpu.VMEM((NP,), _f32), pltpu.VMEM((NP,), _f32), pltpu.VMEM((NP,), _f32),
        pltpu.VMEM((NP,), _f32), pltpu.VMEM((NP,), _f32), pltpu.VMEM((NP,), _f32),
        pltpu.VMEM((HCPP,), _i32),
        pltpu.VMEM((_VPT,), _f32), pltpu.VMEM((_VPT,), _f32), pltpu.VMEM((_VPT,), _f32),
        pltpu.VMEM((_VPT,), _f32), pltpu.VMEM((_VPT,), _f32), pltpu.VMEM((_VPT,), _f32),
        pltpu.VMEM((_HPT,), _f32), pltpu.VMEM((_HPT,), _f32),
    ]
    return pl.kernel(_gather_body, out_type=out_type, mesh=mesh,
                     scratch_types=scratch,
                     compiler_params=pltpu.CompilerParams(
                         needs_layout_passes=False),
                     )(idx_t, md_t, vx, vy, vz, nx, ny, nz, hcp)


def _finish_body(md_ref, gc_ref, gxj_ref, gyj_ref, gzj_ref,
                 gnx_ref, gny_ref, gnz_ref, vnx_ref, vny_ref, vnz_ref,
                 xx_ref, xy_ref, xz_ref, ivx_ref, ivy_ref, ivz_ref,
                 hvi_ref, vh_ref, hw_ref, bp_ref, ip_ref, lhp_ref, rhp_ref,
                 out_ref):
    pos = (lax.broadcasted_iota(_i32, (NR, 128), 0) * 128
           + lax.broadcasted_iota(_i32, (NR, 128), 1))
    valid = pos < N
    v2v = jnp.sqrt(md_ref[...] + 1e-12)
    xx = xx_ref[...]
    xy = xy_ref[...]
    xz = xz_ref[...]
    gnx = gnx_ref[...]
    gny = gny_ref[...]
    gnz = gnz_ref[...]
    vnx = vnx_ref[...]
    vny = vny_ref[...]
    vnz = vnz_ref[...]
    ext = (gnx * (gxj_ref[...] - xx) + gny * (gyj_ref[...] - xy)
           + gnz * (gzj_ref[...] - xz)) > 0.0
    isds = jnp.logical_and((pos % 8) == 0, valid)
    inside = jnp.logical_and(isds, jnp.logical_not(ext))

    def mmean(v, m):
        c = jnp.sum(jnp.where(m, 1.0, 0.0))
        s = jnp.sum(jnp.where(m, v, 0.0))
        return jnp.where(c > 0, s / jnp.maximum(c, 1.0), 0.0)

    gc = gc_ref[...]
    wout = 1.0 / (5.0 * gc + 1.0)
    contact = CONTACT_W * mmean(A1 * wout * jnp.tanh(v2v / A2),
                                jnp.logical_and(isds, jnp.logical_not(inside)))
    insidel = INSIDE_W * mmean(B1 * jnp.tanh(v2v / B2), inside)
    ni = jnp.sqrt(vnx * vnx + vny * vny + vnz * vnz)
    nj = jnp.sqrt(gnx * gnx + gny * gny + gnz * gnz)
    ng = (vnx * gnx + vny * gny + vnz * gnz) / ((ni + 1e-8) * (nj + 1e-8))
    angle = ANGLE_W * mmean(1.0 + ng, jnp.logical_and(v2v < 0.01, valid))

    odel = jnp.sqrt((ivx_ref[...] - xx) ** 2 + (ivy_ref[...] - xy) ** 2
                    + (ivz_ref[...] - xz) ** 2 + 1e-12)
    outside = OUTSIDE_W * jnp.sum(jnp.where(valid, odel * (2.0 * gc) ** 2, 0.0))

    hpos = (lax.broadcasted_iota(_i32, (16, 128), 0) * 128
            + lax.broadcasted_iota(_i32, (16, 128), 1))
    lmask = hpos < HA
    rmask = jnp.logical_and(hpos >= HA, hpos < 2 * HA)
    hv = hvi_ref[...] > 0.5
    nhv = jnp.logical_not(hv)
    v2vh = jnp.sqrt(vh_ref[...] + 1e-12)
    w = -0.1 * hw_ref[...] + 1.0
    vout = w * (C1 * jnp.tanh(v2vh / C2))
    vin = D1 * jnp.tanh(v2vh / D2)
    hco = mmean(vout, jnp.logical_and(lmask, nhv)) + \
          mmean(vout, jnp.logical_and(rmask, nhv))
    hci = mmean(vin, jnp.logical_and(lmask, hv)) + \
          mmean(vin, jnp.logical_and(rmask, hv))
    hand = HCP_W * (hci + hco)

    bp = bp_ref[...] - ip_ref[...]
    pose = POSE_W * jnp.sum(bp * bp)
    hpp = HPP_W * (jnp.sum(lhp_ref[...] ** 2) + jnp.sum(rhp_ref[...] ** 2))
    total = contact + insidel + outside + angle + pose + hpp + hand
    out_ref[...] = total[None, None]


def _finish_call(*args):
    return pl.pallas_call(
        _finish_body,
        out_shape=jax.ShapeDtypeStruct((1, 1), _f32),
    )(*args)


@jax.jit
def kernel(vertices, body_pose, left_hand_pose, right_hand_pose, init_verts,
           init_pose, geodist, hand_contact_prior_weights, faces, ds,
           hand_contact_prior, init_verts_in_contact):
    x = vertices[0]
    vx = jnp.pad(x[:, 0], (0, NP - N))
    vy = jnp.pad(x[:, 1], (0, NP - N))
    vz = jnp.pad(x[:, 2], (0, NP - N))
    f0 = jnp.pad(faces[:, 0], (0, FP - F), constant_values=SINK)
    f1 = jnp.pad(faces[:, 1], (0, FP - F), constant_values=SINK)
    f2 = jnp.pad(faces[:, 2], (0, FP - F), constant_values=SINK)
    ivc = jnp.pad(init_verts_in_contact, (0, NCP - 400), constant_values=SINK)
    hcp = jnp.pad(hand_contact_prior, (0, HCPP - 2 * HA))

    cm2 = _cmask_call(ivc)
    vnx2, vny2, vnz2 = _normals_call(vx, vy, vz, f0, f1, f2)

    sq = jnp.sum(x * x, axis=1)
    zeros = jnp.zeros((NP,), _f32)
    a = jnp.stack([vx, vy, vz, zeros, zeros, zeros, zeros, zeros], 1)
    b = a.T[:, :N]
    cm = cm2.reshape(NP)[:N][None, :]
    mind2, idx, gc = _stream_call(a, b, sq.reshape(N, 1), sq[None, :],
                                  geodist, cm)

    idx_t = jnp.pad(idx[:, 0], (0, NP - N))
    md_t = jnp.pad(mind2[:, 0], (0, NP - N))
    gxj, gyj, gzj, gnx, gny, gnz, hvi, vh = _gather_call(
        idx_t, md_t, vx, vy, vz,
        vnx2.reshape(NP), vny2.reshape(NP), vnz2.reshape(NP), hcp)

    r2 = lambda v: v.reshape(NR, 128)
    gc_t = jnp.pad(gc[:, 0], (0, NP - N))
    ivx = jnp.pad(init_verts[0, :, 0], (0, NP - N))
    ivy = jnp.pad(init_verts[0, :, 1], (0, NP - N))
    ivz = jnp.pad(init_verts[0, :, 2], (0, NP - N))
    hw = jnp.pad(hand_contact_prior_weights, (0, HCPP - 2 * HA)).reshape(16, 128)
    bp = jnp.pad(body_pose[0], (0, 128 - 69))[None, :]
    ip = jnp.pad(init_pose[0], (0, 128 - 69))[None, :]
    lhp = jnp.pad(left_hand_pose[0], (0, 128 - 45))[None, :]
    rhp = jnp.pad(right_hand_pose[0], (0, 128 - 45))[None, :]

    out = _finish_call(r2(md_t), r2(gc_t), r2(gxj), r2(gyj), r2(gzj),
                       r2(gnx), r2(gny), r2(gnz), vnx2, vny2, vnz2,
                       r2(vx), r2(vy), r2(vz), r2(ivx), r2(ivy), r2(ivz),
                       hvi.reshape(16, 128), vh.reshape(16, 128), hw,
                       bp, ip, lhp, rhp)
    return out[0, 0]


# speedup vs baseline: 1.0408x; 1.0401x over previous
import functools

import jax
import jax.numpy as jnp
from jax import lax
from jax.experimental import pallas as pl
from jax.experimental.pallas import tpu as pltpu
from jax.experimental.pallas import tpu_sc as plsc

N = 6890
NP = 8192
NR = 64
F = 13776
FP = 13824
FPT = 864
HA = 778
HCPP = 2048
NCP = 512
SINK = 6900

INSIDE_W = 0.5
OUTSIDE_W = 0.005
CONTACT_W = 10.0
HCP_W = 1.0
POSE_W = 0.01
HPP_W = 0.01
ANGLE_W = 0.1
A1, A2 = 0.04, 0.04
B1, B2 = 0.07, 0.06
C1, C2 = 0.01, 0.01
D1, D2 = 0.023, 0.02
GEO_THRES = 0.3

_f32 = jnp.float32
_i32 = jnp.int32


def _zero2d(ref, nrows):
    def body(i, _):
        r = i // 8
        c = (i % 8) * 16
        ref[r, pl.ds(c, 16)] = jnp.zeros((16,), _f32)
        return 0
    lax.fori_loop(0, nrows * 8, body, 0)


def _cmask_body(ivc_h, cm_h, ivcl, acc):
    tid = lax.axis_index("s")

    @pl.when(tid == 0)
    def _():
        pltpu.sync_copy(ivc_h, ivcl)
        _zero2d(acc, NR)

        def cbody(k, _):
            ii = ivcl[pl.ds(k * 16, 16)]
            r = lax.shift_right_logical(ii, 7)
            c = lax.bitwise_and(ii, 127)
            plsc.store_scatter(acc, [r, c], jnp.ones((16,), _f32))
            return 0
        lax.fori_loop(0, NCP // 16, cbody, 0)
        pltpu.sync_copy(acc, cm_h)


def _cmask_call(ivc):
    mesh = plsc.VectorSubcoreMesh(core_axis_name="c", subcore_axis_name="s",
                                  num_cores=1)
    return pl.kernel(_cmask_body,
                     out_type=jax.ShapeDtypeStruct((NR, 128), _f32),
                     mesh=mesh,
                     scratch_types=[pltpu.VMEM((NCP,), _i32),
                                    pltpu.VMEM((NR, 128), _f32)],
                     compiler_params=pltpu.CompilerParams(
                         needs_layout_passes=False),
                     )(ivc)


def _normals_body(vx_h, vy_h, vz_h, f0_h, f1_h, f2_h,
                  vnx_h, vny_h, vnz_h,
                  vxl, vyl, vzl, f0l, f1l, f2l,
                  accx, accy, accz, rbuf,
                  svnx, svny, svnz):
    tid = lax.axis_index("s")
    pltpu.sync_copy(vx_h, vxl)
    pltpu.sync_copy(vy_h, vyl)
    pltpu.sync_copy(vz_h, vzl)
    pltpu.sync_copy(f0_h.at[pl.ds(tid * FPT, FPT)], f0l)
    pltpu.sync_copy(f1_h.at[pl.ds(tid * FPT, FPT)], f1l)
    pltpu.sync_copy(f2_h.at[pl.ds(tid * FPT, FPT)], f2l)

    _zero2d(accx, NR)
    _zero2d(accy, NR)
    _zero2d(accz, NR)

    def fbody(b, _):
        s = b * 16
        i0 = f0l[pl.ds(s, 16)]
        i1 = f1l[pl.ds(s, 16)]
        i2 = f2l[pl.ds(s, 16)]
        v0x = plsc.load_gather(vxl, [i0])
        v0y = plsc.load_gather(vyl, [i0])
        v0z = plsc.load_gather(vzl, [i0])
        v1x = plsc.load_gather(vxl, [i1])
        v1y = plsc.load_gather(vyl, [i1])
        v1z = plsc.load_gather(vzl, [i1])
        v2x = plsc.load_gather(vxl, [i2])
        v2y = plsc.load_gather(vyl, [i2])
        v2z = plsc.load_gather(vzl, [i2])
        e1x = v1x - v0x
        e1y = v1y - v0y
        e1z = v1z - v0z
        e2x = v2x - v0x
        e2y = v2y - v0y
        e2z = v2z - v0z
        fnx = e1y * e2z - e1z * e2y
        fny = e1z * e2x - e1x * e2z
        fnz = e1x * e2y - e1y * e2x
        for ii in (i0, i1, i2):
            r = lax.shift_right_logical(ii, 7)
            c = lax.bitwise_and(ii, 127)
            plsc.addupdate_scatter(accx, [r, c], fnx)
            plsc.addupdate_scatter(accy, [r, c], fny)
            plsc.addupdate_scatter(accz, [r, c], fnz)
        return 0
    lax.fori_loop(0, FPT // 16, fbody, 0)

    pltpu.sync_copy(accx, svnx.at[tid])
    pltpu.sync_copy(accy, svny.at[tid])
    pltpu.sync_copy(accz, svnz.at[tid])
    plsc.subcore_barrier()

    rpt = NR // 16
    for src, dst in ((svnx, vnx_h), (svny, vny_h), (svnz, vnz_h)):
        for s in range(16):
            pltpu.sync_copy(src.at[s, pl.ds(tid * rpt, rpt)], rbuf.at[s])

        def rbody(i, _):
            r = i // 8
            c = (i % 8) * 16
            tot = rbuf[0, r, pl.ds(c, 16)]
            for s in range(1, 16):
                tot = tot + rbuf[s, r, pl.ds(c, 16)]
            accx[r, pl.ds(c, 16)] = tot
            return 0
        lax.fori_loop(0, rpt * 8, rbody, 0)
        pltpu.sync_copy(accx.at[pl.ds(0, rpt)], dst.at[pl.ds(tid * rpt, rpt)])


def _normals_call(vx, vy, vz, f0, f1, f2):
    mesh = plsc.VectorSubcoreMesh(core_axis_name="c", subcore_axis_name="s",
                                  num_cores=1)
    out_type = [jax.ShapeDtypeStruct((NR, 128), _f32) for _ in range(3)]
    scratch = [
        pltpu.VMEM((NP,), _f32), pltpu.VMEM((NP,), _f32), pltpu.VMEM((NP,), _f32),
        pltpu.VMEM((FPT,), _i32), pltpu.VMEM((FPT,), _i32), pltpu.VMEM((FPT,), _i32),
        pltpu.VMEM((NR, 128), _f32), pltpu.VMEM((NR, 128), _f32),
        pltpu.VMEM((NR, 128), _f32),
        pltpu.VMEM((16, NR // 16, 128), _f32),
        pltpu.VMEM_SHARED((16, NR, 128), _f32),
        pltpu.VMEM_SHARED((16, NR, 128), _f32),
        pltpu.VMEM_SHARED((16, NR, 128), _f32),
    ]
    return pl.kernel(_normals_body, out_type=out_type, mesh=mesh,
                     scratch_types=scratch,
                     compiler_params=pltpu.CompilerParams(
                         needs_layout_passes=False),
                     )(vx, vy, vz, f0, f1, f2)


RB = 256
_GRID = 27


_RBR = RB // 128


def _stream_body(a_ref, b_ref, g_ref, cm_ref, min_ref, idx_ref, gc_ref):
    a = a_ref[...]
    b = b_ref[...]
    p = lax.dot_general(a, b, (((1,), (0,)), ((), ())),
                        preferred_element_type=_f32)
    sqr = jnp.sum(a * a, axis=1)
    sqc = jnp.sum(b * b, axis=0)
    d2 = sqr[:, None] + sqc[None, :] - 2.0 * p
    d2 = jnp.maximum(d2, 0.0)
    g = g_ref[...]
    d2m = jnp.where(g < GEO_THRES, 1e10, d2)
    m = jnp.min(d2m, axis=1)
    iota = lax.broadcasted_iota(_i32, d2m.shape, 1)
    idx = jnp.min(jnp.where(d2m == m[:, None], iota, N), axis=1)
    gc = jnp.min(jnp.where(cm_ref[...] > 0.5, g, 1e10), axis=1)
    min_ref[...] = m.reshape(1, _RBR, 128)
    idx_ref[...] = idx.reshape(1, _RBR, 128)
    gc_ref[...] = gc.reshape(1, _RBR, 128)


def _stream_call(a, b, geodist, cm):
    return pl.pallas_call(
        _stream_body,
        grid=(_GRID,),
        in_specs=[
            pl.BlockSpec((RB, 8), lambda i: (i, 0)),
            pl.BlockSpec((8, N), lambda i: (0, 0)),
            pl.BlockSpec((RB, N), lambda i: (i, 0)),
            pl.BlockSpec((1, N), lambda i: (0, 0)),
        ],
        out_specs=[
            pl.BlockSpec((1, _RBR, 128), lambda i: (i, 0, 0)),
            pl.BlockSpec((1, _RBR, 128), lambda i: (i, 0, 0)),
            pl.BlockSpec((1, _RBR, 128), lambda i: (i, 0, 0)),
        ],
        out_shape=[
            jax.ShapeDtypeStruct((_GRID, _RBR, 128), _f32),
            jax.ShapeDtypeStruct((_GRID, _RBR, 128), _i32),
            jax.ShapeDtypeStruct((_GRID, _RBR, 128), _f32),
        ],
    )(a, b, geodist, cm)


_VPT = NP // 32
_HPT = HCPP // 32


def _gather_body(idx_h, md_h, vx_h, vy_h, vz_h, nx_h, ny_h, nz_h, hcp_h,
                 gxj_h, gyj_h, gzj_h, gnx_h, gny_h, gnz_h, hvi_h, vh_h,
                 idxl, mdl, vxl, vyl, vzl, nxl, nyl, nzl, hcpl,
                 ox, oy, oz, onx, ony, onz, ohv, ovh):
    nc = 2
    wid = lax.axis_index("s") * nc + lax.axis_index("c")
    pltpu.sync_copy(idx_h, idxl)
    pltpu.sync_copy(md_h, mdl)
    pltpu.sync_copy(vx_h, vxl)
    pltpu.sync_copy(vy_h, vyl)
    pltpu.sync_copy(vz_h, vzl)
    pltpu.sync_copy(nx_h, nxl)
    pltpu.sync_copy(ny_h, nyl)
    pltpu.sync_copy(nz_h, nzl)
    pltpu.sync_copy(hcp_h, hcpl)

    def _rc(ii):
        ii = lax.bitwise_and(ii, NP - 1)
        return [lax.shift_right_logical(ii, 7), lax.bitwise_and(ii, 127)]

    def vbody(b, _):
        r = wid * 2 + b // 8
        c = (b % 8) * 16
        jv = _rc(idxl[r, pl.ds(c, 16)])
        ro = b // 8
        ox[ro, pl.ds(c, 16)] = plsc.load_gather(vxl, jv)
        oy[ro, pl.ds(c, 16)] = plsc.load_gather(vyl, jv)
        oz[ro, pl.ds(c, 16)] = plsc.load_gather(vzl, jv)
        onx[ro, pl.ds(c, 16)] = plsc.load_gather(nxl, jv)
        ony[ro, pl.ds(c, 16)] = plsc.load_gather(nyl, jv)
        onz[ro, pl.ds(c, 16)] = plsc.load_gather(nzl, jv)
        return 0
    lax.fori_loop(0, _VPT // 16, vbody, 0)

    def hbody(b, _):
        s = wid * _HPT + b * 16
        hh = hcpl[pl.ds(s, 16)]
        hrc = _rc(hh)
        jh = _rc(plsc.load_gather(idxl, hrc))
        xhx = plsc.load_gather(vxl, hrc)
        xhy = plsc.load_gather(vyl, hrc)
        xhz = plsc.load_gather(vzl, hrc)
        xjx = plsc.load_gather(vxl, jh)
        xjy = plsc.load_gather(vyl, jh)
        xjz = plsc.load_gather(vzl, jh)
        njx = plsc.load_gather(nxl, jh)
        njy = plsc.load_gather(nyl, jh)
        njz = plsc.load_gather(nzl, jh)
        md = plsc.load_gather(mdl, hrc)
        ext = (njx * (xjx - xhx) + njy * (xjy - xhy) + njz * (xjz - xhz)) > 0.0
        isds = lax.bitwise_and(hh, 7) == 0
        hvi = jnp.where(jnp.logical_and(isds, jnp.logical_not(ext)),
                        jnp.ones((16,), _f32), jnp.zeros((16,), _f32))
        o = b * 16
        ohv[pl.ds(o, 16)] = hvi
        ovh[pl.ds(o, 16)] = md
        return 0
    lax.fori_loop(0, _HPT // 16, hbody, 0)

    pltpu.sync_copy(ox, gxj_h.at[pl.ds(wid * 2, 2)])
    pltpu.sync_copy(oy, gyj_h.at[pl.ds(wid * 2, 2)])
    pltpu.sync_copy(oz, gzj_h.at[pl.ds(wid * 2, 2)])
    pltpu.sync_copy(onx, gnx_h.at[pl.ds(wid * 2, 2)])
    pltpu.sync_copy(ony, gny_h.at[pl.ds(wid * 2, 2)])
    pltpu.sync_copy(onz, gnz_h.at[pl.ds(wid * 2, 2)])
    pltpu.sync_copy(ohv, hvi_h.at[pl.ds(wid * _HPT, _HPT)])
    pltpu.sync_copy(ovh, vh_h.at[pl.ds(wid * _HPT, _HPT)])


def _gather_call(idx_t, md_t, vx, vy, vz, nx, ny, nz, hcp):
    mesh = plsc.VectorSubcoreMesh(core_axis_name="c", subcore_axis_name="s",
                                  num_cores=2)
    out_type = [jax.ShapeDtypeStruct((NR, 128), _f32) for _ in range(6)] + \
               [jax.ShapeDtypeStruct((HCPP,), _f32),
                jax.ShapeDtypeStruct((HCPP,), _f32)]
    t2f = pltpu.VMEM((NR, 128), _f32)
    scratch = [
        pltpu.VMEM((NR, 128), _i32), t2f,
        t2f, t2f, t2f,
        t2f, t2f, t2f,
        pltpu.VMEM((HCPP,), _i32),
        pltpu.VMEM((2, 128), _f32), pltpu.VMEM((2, 128), _f32),
        pltpu.VMEM((2, 128), _f32), pltpu.VMEM((2, 128), _f32),
        pltpu.VMEM((2, 128), _f32), pltpu.VMEM((2, 128), _f32),
        pltpu.VMEM((_HPT,), _f32), pltpu.VMEM((_HPT,), _f32),
    ]
    return pl.kernel(_gather_body, out_type=out_type, mesh=mesh,
                     scratch_types=scratch,
                     compiler_params=pltpu.CompilerParams(
                         needs_layout_passes=False),
                     )(idx_t, md_t, vx, vy, vz, nx, ny, nz, hcp)


def _finish_body(md_ref, gc_ref, gxj_ref, gyj_ref, gzj_ref,
                 gnx_ref, gny_ref, gnz_ref, vnx_ref, vny_ref, vnz_ref,
                 xx_ref, xy_ref, xz_ref, ivx_ref, ivy_ref, ivz_ref,
                 hvi_ref, vh_ref, hw_ref, bp_ref, ip_ref, lhp_ref, rhp_ref,
                 out_ref):
    pos = (lax.broadcasted_iota(_i32, (NR, 128), 0) * 128
           + lax.broadcasted_iota(_i32, (NR, 128), 1))
    valid = pos < N
    v2v = jnp.sqrt(md_ref[...] + 1e-12)
    xx = xx_ref[...]
    xy = xy_ref[...]
    xz = xz_ref[...]
    gnx = gnx_ref[...]
    gny = gny_ref[...]
    gnz = gnz_ref[...]
    vnx = vnx_ref[...]
    vny = vny_ref[...]
    vnz = vnz_ref[...]
    ext = (gnx * (gxj_ref[...] - xx) + gny * (gyj_ref[...] - xy)
           + gnz * (gzj_ref[...] - xz)) > 0.0
    isds = jnp.logical_and((pos % 8) == 0, valid)
    inside = jnp.logical_and(isds, jnp.logical_not(ext))

    def mmean(v, m):
        c = jnp.sum(jnp.where(m, 1.0, 0.0))
        s = jnp.sum(jnp.where(m, v, 0.0))
        return jnp.where(c > 0, s / jnp.maximum(c, 1.0), 0.0)

    gc = gc_ref[...]
    wout = 1.0 / (5.0 * gc + 1.0)
    contact = CONTACT_W * mmean(A1 * wout * jnp.tanh(v2v / A2),
                                jnp.logical_and(isds, jnp.logical_not(inside)))
    insidel = INSIDE_W * mmean(B1 * jnp.tanh(v2v / B2), inside)
    ni = jnp.sqrt(vnx * vnx + vny * vny + vnz * vnz)
    nj = jnp.sqrt(gnx * gnx + gny * gny + gnz * gnz)
    ng = (vnx * gnx + vny * gny + vnz * gnz) / ((ni + 1e-8) * (nj + 1e-8))
    angle = ANGLE_W * mmean(1.0 + ng, jnp.logical_and(v2v < 0.01, valid))

    odel = jnp.sqrt((ivx_ref[...] - xx) ** 2 + (ivy_ref[...] - xy) ** 2
                    + (ivz_ref[...] - xz) ** 2 + 1e-12)
    outside = OUTSIDE_W * jnp.sum(jnp.where(valid, odel * (2.0 * gc) ** 2, 0.0))

    hpos = (lax.broadcasted_iota(_i32, (16, 128), 0) * 128
            + lax.broadcasted_iota(_i32, (16, 128), 1))
    lmask = hpos < HA
    rmask = jnp.logical_and(hpos >= HA, hpos < 2 * HA)
    hv = hvi_ref[...] > 0.5
    nhv = jnp.logical_not(hv)
    v2vh = jnp.sqrt(vh_ref[...] + 1e-12)
    w = -0.1 * hw_ref[...] + 1.0
    vout = w * (C1 * jnp.tanh(v2vh / C2))
    vin = D1 * jnp.tanh(v2vh / D2)
    hco = mmean(vout, jnp.logical_and(lmask, nhv)) + \
          mmean(vout, jnp.logical_and(rmask, nhv))
    hci = mmean(vin, jnp.logical_and(lmask, hv)) + \
          mmean(vin, jnp.logical_and(rmask, hv))
    hand = HCP_W * (hci + hco)

    bp = bp_ref[...] - ip_ref[...]
    pose = POSE_W * jnp.sum(bp * bp)
    hpp = HPP_W * (jnp.sum(lhp_ref[...] ** 2) + jnp.sum(rhp_ref[...] ** 2))
    total = contact + insidel + outside + angle + pose + hpp + hand
    out_ref[...] = total[None, None]


def _finish_call(*args):
    return pl.pallas_call(
        _finish_body,
        out_shape=jax.ShapeDtypeStruct((1, 1), _f32),
    )(*args)


@jax.jit
def kernel(vertices, body_pose, left_hand_pose, right_hand_pose, init_verts,
           init_pose, geodist, hand_contact_prior_weights, faces, ds,
           hand_contact_prior, init_verts_in_contact):
    x = vertices[0]
    vx = jnp.pad(x[:, 0], (0, NP - N))
    vy = jnp.pad(x[:, 1], (0, NP - N))
    vz = jnp.pad(x[:, 2], (0, NP - N))
    f0 = jnp.pad(faces[:, 0], (0, FP - F), constant_values=SINK)
    f1 = jnp.pad(faces[:, 1], (0, FP - F), constant_values=SINK)
    f2 = jnp.pad(faces[:, 2], (0, FP - F), constant_values=SINK)
    ivc = jnp.pad(init_verts_in_contact, (0, NCP - 400), constant_values=SINK)
    hcp = jnp.pad(hand_contact_prior, (0, HCPP - 2 * HA))

    cm2 = _cmask_call(ivc)
    vnx2, vny2, vnz2 = _normals_call(vx, vy, vz, f0, f1, f2)

    zeros = jnp.zeros((NP,), _f32)
    a = jnp.stack([vx, vy, vz, zeros, zeros, zeros, zeros, zeros], 1)
    b = a.T[:, :N]
    cm = cm2.reshape(NP)[:N][None, :]
    mind2_3, idx_3, gc_3 = _stream_call(a, b, geodist, cm)
    pad10 = lambda v: jnp.pad(v.reshape(_GRID * _RBR, 128),
                              ((0, NR - _GRID * _RBR), (0, 0)))
    mind2 = pad10(mind2_3)
    idx = pad10(idx_3)
    gc = pad10(gc_3)

    r2 = lambda v: v.reshape(NR, 128)
    gxj, gyj, gzj, gnx, gny, gnz, hvi, vh = _gather_call(
        idx, mind2, r2(vx), r2(vy), r2(vz),
        vnx2, vny2, vnz2, hcp)

    ivx = jnp.pad(init_verts[0, :, 0], (0, NP - N))
    ivy = jnp.pad(init_verts[0, :, 1], (0, NP - N))
    ivz = jnp.pad(init_verts[0, :, 2], (0, NP - N))
    hw = jnp.pad(hand_contact_prior_weights, (0, HCPP - 2 * HA)).reshape(16, 128)
    bp = jnp.pad(body_pose[0], (0, 128 - 69))[None, :]
    ip = jnp.pad(init_pose[0], (0, 128 - 69))[None, :]
    lhp = jnp.pad(left_hand_pose[0], (0, 128 - 45))[None, :]
    rhp = jnp.pad(right_hand_pose[0], (0, 128 - 45))[None, :]

    out = _finish_call(mind2, gc, gxj, gyj, gzj,
                       gnx, gny, gnz, vnx2, vny2, vnz2,
                       r2(vx), r2(vy), r2(vz), r2(ivx), r2(ivy), r2(ivz),
                       hvi.reshape(16, 128), vh.reshape(16, 128), hw,
                       bp, ip, lhp, rhp)
    return out[0, 0]


# speedup vs baseline: 1.0674x; 1.0256x over previous
import functools

import jax
import jax.numpy as jnp
from jax import lax
from jax.experimental import pallas as pl
from jax.experimental.pallas import tpu as pltpu
from jax.experimental.pallas import tpu_sc as plsc

N = 6890
NP = 8192
NR = 64
F = 13776
FP = 13824
FPT = 864
HA = 778
HCPP = 2048
NCP = 512
SINK = 6900

INSIDE_W = 0.5
OUTSIDE_W = 0.005
CONTACT_W = 10.0
HCP_W = 1.0
POSE_W = 0.01
HPP_W = 0.01
ANGLE_W = 0.1
A1, A2 = 0.04, 0.04
B1, B2 = 0.07, 0.06
C1, C2 = 0.01, 0.01
D1, D2 = 0.023, 0.02
GEO_THRES = 0.3

_f32 = jnp.float32
_i32 = jnp.int32


def _zero2d(ref, nrows):
    def body(i, _):
        r = i // 8
        c = (i % 8) * 16
        ref[r, pl.ds(c, 16)] = jnp.zeros((16,), _f32)
        return 0
    lax.fori_loop(0, nrows * 8, body, 0)


def _cmask_body(ivc_h, cm_h, ivcl, acc):
    tid = lax.axis_index("s")

    @pl.when(tid == 0)
    def _():
        pltpu.sync_copy(ivc_h, ivcl)
        _zero2d(acc, NR)

        def cbody(k, _):
            ii = ivcl[pl.ds(k * 16, 16)]
            r = lax.shift_right_logical(ii, 7)
            c = lax.bitwise_and(ii, 127)
            plsc.store_scatter(acc, [r, c], jnp.ones((16,), _f32))
            return 0
        lax.fori_loop(0, NCP // 16, cbody, 0)
        pltpu.sync_copy(acc, cm_h)


def _cmask_call(ivc):
    mesh = plsc.VectorSubcoreMesh(core_axis_name="c", subcore_axis_name="s",
                                  num_cores=1)
    return pl.kernel(_cmask_body,
                     out_type=jax.ShapeDtypeStruct((NR, 128), _f32),
                     mesh=mesh,
                     scratch_types=[pltpu.VMEM((NCP,), _i32),
                                    pltpu.VMEM((NR, 128), _f32)],
                     compiler_params=pltpu.CompilerParams(
                         needs_layout_passes=False),
                     )(ivc)


def _normals_body(vx_h, vy_h, vz_h, f0_h, f1_h, f2_h,
                  vnx_h, vny_h, vnz_h,
                  vxl, vyl, vzl, f0l, f1l, f2l,
                  accx, accy, accz, rbuf,
                  svnx, svny, svnz):
    tid = lax.axis_index("s")
    pltpu.sync_copy(vx_h, vxl)
    pltpu.sync_copy(vy_h, vyl)
    pltpu.sync_copy(vz_h, vzl)
    pltpu.sync_copy(f0_h.at[pl.ds(tid * FPT, FPT)], f0l)
    pltpu.sync_copy(f1_h.at[pl.ds(tid * FPT, FPT)], f1l)
    pltpu.sync_copy(f2_h.at[pl.ds(tid * FPT, FPT)], f2l)

    _zero2d(accx, NR)
    _zero2d(accy, NR)
    _zero2d(accz, NR)

    def fbody(b, _):
        s = b * 16
        i0 = f0l[pl.ds(s, 16)]
        i1 = f1l[pl.ds(s, 16)]
        i2 = f2l[pl.ds(s, 16)]
        v0x = plsc.load_gather(vxl, [i0])
        v0y = plsc.load_gather(vyl, [i0])
        v0z = plsc.load_gather(vzl, [i0])
        v1x = plsc.load_gather(vxl, [i1])
        v1y = plsc.load_gather(vyl, [i1])
        v1z = plsc.load_gather(vzl, [i1])
        v2x = plsc.load_gather(vxl, [i2])
        v2y = plsc.load_gather(vyl, [i2])
        v2z = plsc.load_gather(vzl, [i2])
        e1x = v1x - v0x
        e1y = v1y - v0y
        e1z = v1z - v0z
        e2x = v2x - v0x
        e2y = v2y - v0y
        e2z = v2z - v0z
        fnx = e1y * e2z - e1z * e2y
        fny = e1z * e2x - e1x * e2z
        fnz = e1x * e2y - e1y * e2x
        for ii in (i0, i1, i2):
            r = lax.shift_right_logical(ii, 7)
            c = lax.bitwise_and(ii, 127)
            plsc.addupdate_scatter(accx, [r, c], fnx)
            plsc.addupdate_scatter(accy, [r, c], fny)
            plsc.addupdate_scatter(accz, [r, c], fnz)
        return 0
    lax.fori_loop(0, FPT // 16, fbody, 0)

    pltpu.sync_copy(accx, svnx.at[tid])
    pltpu.sync_copy(accy, svny.at[tid])
    pltpu.sync_copy(accz, svnz.at[tid])
    plsc.subcore_barrier()

    rpt = NR // 16
    for src, dst in ((svnx, vnx_h), (svny, vny_h), (svnz, vnz_h)):
        for s in range(16):
            pltpu.sync_copy(src.at[s, pl.ds(tid * rpt, rpt)], rbuf.at[s])

        def rbody(i, _):
            r = i // 8
            c = (i % 8) * 16
            tot = rbuf[0, r, pl.ds(c, 16)]
            for s in range(1, 16):
                tot = tot + rbuf[s, r, pl.ds(c, 16)]
            accx[r, pl.ds(c, 16)] = tot
            return 0
        lax.fori_loop(0, rpt * 8, rbody, 0)
        pltpu.sync_copy(accx.at[pl.ds(0, rpt)], dst.at[pl.ds(tid * rpt, rpt)])


def _normals_call(vx, vy, vz, f0, f1, f2):
    mesh = plsc.VectorSubcoreMesh(core_axis_name="c", subcore_axis_name="s",
                                  num_cores=1)
    out_type = [jax.ShapeDtypeStruct((NR, 128), _f32) for _ in range(3)]
    scratch = [
        pltpu.VMEM((NP,), _f32), pltpu.VMEM((NP,), _f32), pltpu.VMEM((NP,), _f32),
        pltpu.VMEM((FPT,), _i32), pltpu.VMEM((FPT,), _i32), pltpu.VMEM((FPT,), _i32),
        pltpu.VMEM((NR, 128), _f32), pltpu.VMEM((NR, 128), _f32),
        pltpu.VMEM((NR, 128), _f32),
        pltpu.VMEM((16, NR // 16, 128), _f32),
        pltpu.VMEM_SHARED((16, NR, 128), _f32),
        pltpu.VMEM_SHARED((16, NR, 128), _f32),
        pltpu.VMEM_SHARED((16, NR, 128), _f32),
    ]
    return pl.kernel(_normals_body, out_type=out_type, mesh=mesh,
                     scratch_types=scratch,
                     compiler_params=pltpu.CompilerParams(
                         needs_layout_passes=False),
                     )(vx, vy, vz, f0, f1, f2)


RB = 256
_GRID = 27


_RBR = RB // 128


def _stream_body(a_ref, b_ref, g_ref, cm_ref, min_ref, idx_ref, gc_ref):
    a = a_ref[...]
    b = b_ref[...]
    p = lax.dot_general(a, b, (((1,), (0,)), ((), ())),
                        preferred_element_type=_f32)
    aa = a * a
    sqr = (aa[:, 0:1] + aa[:, 1:2]) + aa[:, 2:3]
    bb = b * b
    sqc = (bb[0:1, :] + bb[1:2, :]) + bb[2:3, :]
    d2 = sqr + sqc - 2.0 * p
    d2 = jnp.maximum(d2, 0.0)
    g = g_ref[...]
    d2m = jnp.where(g < GEO_THRES, 1e10, d2)
    m = jnp.min(d2m, axis=1)
    iota = lax.broadcasted_iota(_i32, d2m.shape, 1)
    idx = jnp.min(jnp.where(d2m == m[:, None], iota, N), axis=1)
    gc = jnp.min(jnp.where(cm_ref[...] > 0.5, g, 1e10), axis=1)
    min_ref[...] = m.reshape(1, _RBR, 128)
    idx_ref[...] = idx.reshape(1, _RBR, 128)
    gc_ref[...] = gc.reshape(1, _RBR, 128)


def _stream_call(a, b, geodist, cm):
    return pl.pallas_call(
        _stream_body,
        grid=(_GRID,),
        in_specs=[
            pl.BlockSpec((RB, 8), lambda i: (i, 0)),
            pl.BlockSpec((8, N), lambda i: (0, 0)),
            pl.BlockSpec((RB, N), lambda i: (i, 0)),
            pl.BlockSpec((1, N), lambda i: (0, 0)),
        ],
        out_specs=[
            pl.BlockSpec((1, _RBR, 128), lambda i: (i, 0, 0)),
            pl.BlockSpec((1, _RBR, 128), lambda i: (i, 0, 0)),
            pl.BlockSpec((1, _RBR, 128), lambda i: (i, 0, 0)),
        ],
        out_shape=[
            jax.ShapeDtypeStruct((_GRID, _RBR, 128), _f32),
            jax.ShapeDtypeStruct((_GRID, _RBR, 128), _i32),
            jax.ShapeDtypeStruct((_GRID, _RBR, 128), _f32),
        ],
    )(a, b, geodist, cm)


_VPT = NP // 32
_HPT = HCPP // 32


def _gather_body(idx_h, md_h, vx_h, vy_h, vz_h, nx_h, ny_h, nz_h, hcp_h,
                 gxj_h, gyj_h, gzj_h, gnx_h, gny_h, gnz_h, hvi_h, vh_h,
                 idxl, mdl, vxl, vyl, vzl, nxl, nyl, nzl, hcpl,
                 ox, oy, oz, onx, ony, onz, ohv, ovh, sem_in, sem_out):
    nc = 2
    wid = lax.axis_index("s") * nc + lax.axis_index("c")
    hs = [pltpu.async_copy(s, t, sem_in)
          for s, t in ((idx_h, idxl), (md_h, mdl), (vx_h, vxl), (vy_h, vyl),
                       (vz_h, vzl), (nx_h, nxl), (ny_h, nyl), (nz_h, nzl),
                       (hcp_h, hcpl))]
    for h in hs:
        h.wait()

    def _rc(ii):
        ii = lax.bitwise_and(ii, NP - 1)
        return [lax.shift_right_logical(ii, 7), lax.bitwise_and(ii, 127)]

    def vbody(b, _):
        r = wid * 2 + b // 8
        c = (b % 8) * 16
        jv = _rc(idxl[r, pl.ds(c, 16)])
        ro = b // 8
        ox[ro, pl.ds(c, 16)] = plsc.load_gather(vxl, jv)
        oy[ro, pl.ds(c, 16)] = plsc.load_gather(vyl, jv)
        oz[ro, pl.ds(c, 16)] = plsc.load_gather(vzl, jv)
        onx[ro, pl.ds(c, 16)] = plsc.load_gather(nxl, jv)
        ony[ro, pl.ds(c, 16)] = plsc.load_gather(nyl, jv)
        onz[ro, pl.ds(c, 16)] = plsc.load_gather(nzl, jv)
        return 0
    lax.fori_loop(0, _VPT // 16, vbody, 0)

    def hbody(b, _):
        s = wid * _HPT + b * 16
        hh = hcpl[pl.ds(s, 16)]
        hrc = _rc(hh)
        jh = _rc(plsc.load_gather(idxl, hrc))
        xhx = plsc.load_gather(vxl, hrc)
        xhy = plsc.load_gather(vyl, hrc)
        xhz = plsc.load_gather(vzl, hrc)
        xjx = plsc.load_gather(vxl, jh)
        xjy = plsc.load_gather(vyl, jh)
        xjz = plsc.load_gather(vzl, jh)
        njx = plsc.load_gather(nxl, jh)
        njy = plsc.load_gather(nyl, jh)
        njz = plsc.load_gather(nzl, jh)
        md = plsc.load_gather(mdl, hrc)
        ext = (njx * (xjx - xhx) + njy * (xjy - xhy) + njz * (xjz - xhz)) > 0.0
        isds = lax.bitwise_and(hh, 7) == 0
        hvi = jnp.where(jnp.logical_and(isds, jnp.logical_not(ext)),
                        jnp.ones((16,), _f32), jnp.zeros((16,), _f32))
        o = b * 16
        ohv[pl.ds(o, 16)] = hvi
        ovh[pl.ds(o, 16)] = md
        return 0
    lax.fori_loop(0, _HPT // 16, hbody, 0)

    ho = [pltpu.async_copy(s, t, sem_out)
          for s, t in ((ox, gxj_h.at[pl.ds(wid * 2, 2)]),
                       (oy, gyj_h.at[pl.ds(wid * 2, 2)]),
                       (oz, gzj_h.at[pl.ds(wid * 2, 2)]),
                       (onx, gnx_h.at[pl.ds(wid * 2, 2)]),
                       (ony, gny_h.at[pl.ds(wid * 2, 2)]),
                       (onz, gnz_h.at[pl.ds(wid * 2, 2)]),
                       (ohv, hvi_h.at[pl.ds(wid * _HPT, _HPT)]),
                       (ovh, vh_h.at[pl.ds(wid * _HPT, _HPT)]))]
    for h in ho:
        h.wait()


def _gather_call(idx_t, md_t, vx, vy, vz, nx, ny, nz, hcp):
    mesh = plsc.VectorSubcoreMesh(core_axis_name="c", subcore_axis_name="s",
                                  num_cores=2)
    out_type = [jax.ShapeDtypeStruct((NR, 128), _f32) for _ in range(6)] + \
               [jax.ShapeDtypeStruct((HCPP,), _f32),
                jax.ShapeDtypeStruct((HCPP,), _f32)]
    t2f = pltpu.VMEM((NR, 128), _f32)
    scratch = [
        pltpu.VMEM((NR, 128), _i32), t2f,
        t2f, t2f, t2f,
        t2f, t2f, t2f,
        pltpu.VMEM((HCPP,), _i32),
        pltpu.VMEM((2, 128), _f32), pltpu.VMEM((2, 128), _f32),
        pltpu.VMEM((2, 128), _f32), pltpu.VMEM((2, 128), _f32),
        pltpu.VMEM((2, 128), _f32), pltpu.VMEM((2, 128), _f32),
        pltpu.VMEM((_HPT,), _f32), pltpu.VMEM((_HPT,), _f32),
        pltpu.SemaphoreType.DMA, pltpu.SemaphoreType.DMA,
    ]
    return pl.kernel(_gather_body, out_type=out_type, mesh=mesh,
                     scratch_types=scratch,
                     compiler_params=pltpu.CompilerParams(
                         needs_layout_passes=False),
                     )(idx_t, md_t, vx, vy, vz, nx, ny, nz, hcp)


def _finish_body(md_ref, gc_ref, gxj_ref, gyj_ref, gzj_ref,
                 gnx_ref, gny_ref, gnz_ref, vnx_ref, vny_ref, vnz_ref,
                 xx_ref, xy_ref, xz_ref, ivx_ref, ivy_ref, ivz_ref,
                 hvi_ref, vh_ref, hw_ref, bp_ref, ip_ref, lhp_ref, rhp_ref,
                 out_ref):
    pos = (lax.broadcasted_iota(_i32, (NR, 128), 0) * 128
           + lax.broadcasted_iota(_i32, (NR, 128), 1))
    valid = pos < N
    v2v = jnp.sqrt(md_ref[...] + 1e-12)
    xx = xx_ref[...]
    xy = xy_ref[...]
    xz = xz_ref[...]
    gnx = gnx_ref[...]
    gny = gny_ref[...]
    gnz = gnz_ref[...]
    vnx = vnx_ref[...]
    vny = vny_ref[...]
    vnz = vnz_ref[...]
    ext = (gnx * (gxj_ref[...] - xx) + gny * (gyj_ref[...] - xy)
           + gnz * (gzj_ref[...] - xz)) > 0.0
    isds = jnp.logical_and((pos % 8) == 0, valid)
    inside = jnp.logical_and(isds, jnp.logical_not(ext))

    def mmean(v, m):
        c = jnp.sum(jnp.where(m, 1.0, 0.0))
        s = jnp.sum(jnp.where(m, v, 0.0))
        return jnp.where(c > 0, s / jnp.maximum(c, 1.0), 0.0)

    gc = gc_ref[...]
    wout = 1.0 / (5.0 * gc + 1.0)
    contact = CONTACT_W * mmean(A1 * wout * jnp.tanh(v2v / A2),
                                jnp.logical_and(isds, jnp.logical_not(inside)))
    insidel = INSIDE_W * mmean(B1 * jnp.tanh(v2v / B2), inside)
    ni = jnp.sqrt(vnx * vnx + vny * vny + vnz * vnz)
    nj = jnp.sqrt(gnx * gnx + gny * gny + gnz * gnz)
    ng = (vnx * gnx + vny * gny + vnz * gnz) / ((ni + 1e-8) * (nj + 1e-8))
    angle = ANGLE_W * mmean(1.0 + ng, jnp.logical_and(v2v < 0.01, valid))

    odel = jnp.sqrt((ivx_ref[...] - xx) ** 2 + (ivy_ref[...] - xy) ** 2
                    + (ivz_ref[...] - xz) ** 2 + 1e-12)
    outside = OUTSIDE_W * jnp.sum(jnp.where(valid, odel * (2.0 * gc) ** 2, 0.0))

    hpos = (lax.broadcasted_iota(_i32, (16, 128), 0) * 128
            + lax.broadcasted_iota(_i32, (16, 128), 1))
    lmask = hpos < HA
    rmask = jnp.logical_and(hpos >= HA, hpos < 2 * HA)
    hv = hvi_ref[...] > 0.5
    nhv = jnp.logical_not(hv)
    v2vh = jnp.sqrt(vh_ref[...] + 1e-12)
    w = -0.1 * hw_ref[...] + 1.0
    vout = w * (C1 * jnp.tanh(v2vh / C2))
    vin = D1 * jnp.tanh(v2vh / D2)
    hco = mmean(vout, jnp.logical_and(lmask, nhv)) + \
          mmean(vout, jnp.logical_and(rmask, nhv))
    hci = mmean(vin, jnp.logical_and(lmask, hv)) + \
          mmean(vin, jnp.logical_and(rmask, hv))
    hand = HCP_W * (hci + hco)

    bp = bp_ref[...] - ip_ref[...]
    pose = POSE_W * jnp.sum(bp * bp)
    hpp = HPP_W * (jnp.sum(lhp_ref[...] ** 2) + jnp.sum(rhp_ref[...] ** 2))
    total = contact + insidel + outside + angle + pose + hpp + hand
    out_ref[...] = total[None, None]


def _finish_call(*args):
    return pl.pallas_call(
        _finish_body,
        out_shape=jax.ShapeDtypeStruct((1, 1), _f32),
    )(*args)


@jax.jit
def kernel(vertices, body_pose, left_hand_pose, right_hand_pose, init_verts,
           init_pose, geodist, hand_contact_prior_weights, faces, ds,
           hand_contact_prior, init_verts_in_contact):
    x = vertices[0]
    vx = jnp.pad(x[:, 0], (0, NP - N))
    vy = jnp.pad(x[:, 1], (0, NP - N))
    vz = jnp.pad(x[:, 2], (0, NP - N))
    f0 = jnp.pad(faces[:, 0], (0, FP - F), constant_values=SINK)
    f1 = jnp.pad(faces[:, 1], (0, FP - F), constant_values=SINK)
    f2 = jnp.pad(faces[:, 2], (0, FP - F), constant_values=SINK)
    ivc = jnp.pad(init_verts_in_contact, (0, NCP - 400), constant_values=SINK)
    hcp = jnp.pad(hand_contact_prior, (0, HCPP - 2 * HA))

    cm2 = _cmask_call(ivc)
    vnx2, vny2, vnz2 = _normals_call(vx, vy, vz, f0, f1, f2)

    a = jnp.pad(x, ((0, NP - N), (0, 5)))
    b = jnp.pad(x.T, ((0, 5), (0, 0)))
    cm = cm2.reshape(NP)[:N][None, :]
    mind2_3, idx_3, gc_3 = _stream_call(a, b, geodist, cm)
    pad10 = lambda v: jnp.pad(v.reshape(_GRID * _RBR, 128),
                              ((0, NR - _GRID * _RBR), (0, 0)))
    mind2 = pad10(mind2_3)
    idx = pad10(idx_3)
    gc = pad10(gc_3)

    r2 = lambda v: v.reshape(NR, 128)
    gxj, gyj, gzj, gnx, gny, gnz, hvi, vh = _gather_call(
        idx, mind2, r2(vx), r2(vy), r2(vz),
        vnx2, vny2, vnz2, hcp)

    ivx = jnp.pad(init_verts[0, :, 0], (0, NP - N))
    ivy = jnp.pad(init_verts[0, :, 1], (0, NP - N))
    ivz = jnp.pad(init_verts[0, :, 2], (0, NP - N))
    hw = jnp.pad(hand_contact_prior_weights, (0, HCPP - 2 * HA)).reshape(16, 128)
    bp = jnp.pad(body_pose[0], (0, 128 - 69))[None, :]
    ip = jnp.pad(init_pose[0], (0, 128 - 69))[None, :]
    lhp = jnp.pad(left_hand_pose[0], (0, 128 - 45))[None, :]
    rhp = jnp.pad(right_hand_pose[0], (0, 128 - 45))[None, :]

    out = _finish_call(mind2, gc, gxj, gyj, gzj,
                       gnx, gny, gnz, vnx2, vny2, vnz2,
                       r2(vx), r2(vy), r2(vz), r2(ivx), r2(ivy), r2(ivz),
                       hvi.reshape(16, 128), vh.reshape(16, 128), hw,
                       bp, ip, lhp, rhp)
    return out[0, 0]


# speedup vs baseline: 1.0839x; 1.0154x over previous
import functools

import jax
import jax.numpy as jnp
from jax import lax
from jax.experimental import pallas as pl
from jax.experimental.pallas import tpu as pltpu
from jax.experimental.pallas import tpu_sc as plsc

N = 6890
NP = 8192
NR = 64
F = 13776
FP = 13824
FPT = 864
HA = 778
HCPP = 2048
NCP = 512
SINK = 6900

INSIDE_W = 0.5
OUTSIDE_W = 0.005
CONTACT_W = 10.0
HCP_W = 1.0
POSE_W = 0.01
HPP_W = 0.01
ANGLE_W = 0.1
A1, A2 = 0.04, 0.04
B1, B2 = 0.07, 0.06
C1, C2 = 0.01, 0.01
D1, D2 = 0.023, 0.02
GEO_THRES = 0.3

_f32 = jnp.float32
_i32 = jnp.int32


def _zero2d(ref, nrows):
    def body(i, _):
        r = i // 8
        c = (i % 8) * 16
        ref[r, pl.ds(c, 16)] = jnp.zeros((16,), _f32)
        return 0
    lax.fori_loop(0, nrows * 8, body, 0)


def _cmask_body(ivc_ref, cm_ref):
    iv = ivc_ref[...]
    jj = lax.broadcasted_iota(_i32, (NCP, N), 1)
    hit = jnp.any(iv == jj, axis=0)
    cm_ref[...] = jnp.where(hit, 1.0, 0.0)[None, :]


def _cmask_call(ivc):
    return pl.pallas_call(
        _cmask_body,
        out_shape=jax.ShapeDtypeStruct((1, N), _f32),
    )(ivc.reshape(NCP, 1))


def _normals_body(vx_h, vy_h, vz_h, f0_h, f1_h, f2_h,
                  vnx_h, vny_h, vnz_h,
                  vxl, vyl, vzl, f0l, f1l, f2l,
                  accx, accy, accz, rbuf,
                  svnx, svny, svnz):
    tid = lax.axis_index("s")
    pltpu.sync_copy(vx_h, vxl)
    pltpu.sync_copy(vy_h, vyl)
    pltpu.sync_copy(vz_h, vzl)
    pltpu.sync_copy(f0_h.at[pl.ds(tid * FPT, FPT)], f0l)
    pltpu.sync_copy(f1_h.at[pl.ds(tid * FPT, FPT)], f1l)
    pltpu.sync_copy(f2_h.at[pl.ds(tid * FPT, FPT)], f2l)

    _zero2d(accx, NR)
    _zero2d(accy, NR)
    _zero2d(accz, NR)

    def fbody(b, _):
        s = b * 16
        i0 = f0l[pl.ds(s, 16)]
        i1 = f1l[pl.ds(s, 16)]
        i2 = f2l[pl.ds(s, 16)]
        v0x = plsc.load_gather(vxl, [i0])
        v0y = plsc.load_gather(vyl, [i0])
        v0z = plsc.load_gather(vzl, [i0])
        v1x = plsc.load_gather(vxl, [i1])
        v1y = plsc.load_gather(vyl, [i1])
        v1z = plsc.load_gather(vzl, [i1])
        v2x = plsc.load_gather(vxl, [i2])
        v2y = plsc.load_gather(vyl, [i2])
        v2z = plsc.load_gather(vzl, [i2])
        e1x = v1x - v0x
        e1y = v1y - v0y
        e1z = v1z - v0z
        e2x = v2x - v0x
        e2y = v2y - v0y
        e2z = v2z - v0z
        fnx = e1y * e2z - e1z * e2y
        fny = e1z * e2x - e1x * e2z
        fnz = e1x * e2y - e1y * e2x
        for ii in (i0, i1, i2):
            r = lax.shift_right_logical(ii, 7)
            c = lax.bitwise_and(ii, 127)
            plsc.addupdate_scatter(accx, [r, c], fnx)
            plsc.addupdate_scatter(accy, [r, c], fny)
            plsc.addupdate_scatter(accz, [r, c], fnz)
        return 0
    lax.fori_loop(0, FPT // 16, fbody, 0)

    pltpu.sync_copy(accx, svnx.at[tid])
    pltpu.sync_copy(accy, svny.at[tid])
    pltpu.sync_copy(accz, svnz.at[tid])
    plsc.subcore_barrier()

    rpt = NR // 16
    for src, dst in ((svnx, vnx_h), (svny, vny_h), (svnz, vnz_h)):
        for s in range(16):
            pltpu.sync_copy(src.at[s, pl.ds(tid * rpt, rpt)], rbuf.at[s])

        def rbody(i, _):
            r = i // 8
            c = (i % 8) * 16
            tot = rbuf[0, r, pl.ds(c, 16)]
            for s in range(1, 16):
                tot = tot + rbuf[s, r, pl.ds(c, 16)]
            accx[r, pl.ds(c, 16)] = tot
            return 0
        lax.fori_loop(0, rpt * 8, rbody, 0)
        pltpu.sync_copy(accx.at[pl.ds(0, rpt)], dst.at[pl.ds(tid * rpt, rpt)])


def _normals_call(vx, vy, vz, f0, f1, f2):
    mesh = plsc.VectorSubcoreMesh(core_axis_name="c", subcore_axis_name="s",
                                  num_cores=1)
    out_type = [jax.ShapeDtypeStruct((NR, 128), _f32) for _ in range(3)]
    scratch = [
        pltpu.VMEM((NP,), _f32), pltpu.VMEM((NP,), _f32), pltpu.VMEM((NP,), _f32),
        pltpu.VMEM((FPT,), _i32), pltpu.VMEM((FPT,), _i32), pltpu.VMEM((FPT,), _i32),
        pltpu.VMEM((NR, 128), _f32), pltpu.VMEM((NR, 128), _f32),
        pltpu.VMEM((NR, 128), _f32),
        pltpu.VMEM((16, NR // 16, 128), _f32),
        pltpu.VMEM_SHARED((16, NR, 128), _f32),
        pltpu.VMEM_SHARED((16, NR, 128), _f32),
        pltpu.VMEM_SHARED((16, NR, 128), _f32),
    ]
    return pl.kernel(_normals_body, out_type=out_type, mesh=mesh,
                     scratch_types=scratch,
                     compiler_params=pltpu.CompilerParams(
                         needs_layout_passes=False),
                     )(vx, vy, vz, f0, f1, f2)


RB = 256
_GRID = 27


_RBR = RB // 128


def _stream_body(a_ref, b_ref, g_ref, cm_ref, min_ref, idx_ref, gc_ref):
    a = a_ref[...]
    b = b_ref[...]
    p = lax.dot_general(a, b, (((1,), (0,)), ((), ())),
                        preferred_element_type=_f32)
    aa = a * a
    sqr = (aa[:, 0:1] + aa[:, 1:2]) + aa[:, 2:3]
    bb = b * b
    sqc = (bb[0:1, :] + bb[1:2, :]) + bb[2:3, :]
    d2 = sqr + sqc - 2.0 * p
    d2 = jnp.maximum(d2, 0.0)
    g = g_ref[...]
    d2m = jnp.where(g < GEO_THRES, 1e10, d2)
    m = jnp.min(d2m, axis=1)
    iota = lax.broadcasted_iota(_i32, d2m.shape, 1)
    idx = jnp.min(jnp.where(d2m == m[:, None], iota, N), axis=1)
    gc = jnp.min(jnp.where(cm_ref[...] > 0.5, g, 1e10), axis=1)
    min_ref[...] = m.reshape(1, _RBR, 128)
    idx_ref[...] = idx.reshape(1, _RBR, 128)
    gc_ref[...] = gc.reshape(1, _RBR, 128)


def _stream_call(a, b, geodist, cm):
    return pl.pallas_call(
        _stream_body,
        grid=(_GRID,),
        in_specs=[
            pl.BlockSpec((RB, 8), lambda i: (i, 0)),
            pl.BlockSpec((8, N), lambda i: (0, 0)),
            pl.BlockSpec((RB, N), lambda i: (i, 0)),
            pl.BlockSpec((1, N), lambda i: (0, 0)),
        ],
        out_specs=[
            pl.BlockSpec((1, _RBR, 128), lambda i: (i, 0, 0)),
            pl.BlockSpec((1, _RBR, 128), lambda i: (i, 0, 0)),
            pl.BlockSpec((1, _RBR, 128), lambda i: (i, 0, 0)),
        ],
        out_shape=[
            jax.ShapeDtypeStruct((_GRID, _RBR, 128), _f32),
            jax.ShapeDtypeStruct((_GRID, _RBR, 128), _i32),
            jax.ShapeDtypeStruct((_GRID, _RBR, 128), _f32),
        ],
    )(a, b, geodist, cm)


_VPT = NP // 32
_HPT = HCPP // 32


def _gather_body(idx_h, md_h, vx_h, vy_h, vz_h, nx_h, ny_h, nz_h, hcp_h,
                 gxj_h, gyj_h, gzj_h, gnx_h, gny_h, gnz_h, hvi_h, vh_h,
                 idxl, mdl, vxl, vyl, vzl, nxl, nyl, nzl, hcpl,
                 ox, oy, oz, onx, ony, onz, ohv, ovh, sem_in, sem_out):
    nc = 2
    wid = lax.axis_index("s") * nc + lax.axis_index("c")
    hs = [pltpu.async_copy(s, t, sem_in)
          for s, t in ((idx_h, idxl), (md_h, mdl), (vx_h, vxl), (vy_h, vyl),
                       (vz_h, vzl), (nx_h, nxl), (ny_h, nyl), (nz_h, nzl),
                       (hcp_h, hcpl))]
    for h in hs:
        h.wait()

    def _rc(ii):
        ii = lax.bitwise_and(ii, NP - 1)
        return [lax.shift_right_logical(ii, 7), lax.bitwise_and(ii, 127)]

    def vbody(b, _):
        r = wid * 2 + b // 8
        c = (b % 8) * 16
        jv = _rc(idxl[r, pl.ds(c, 16)])
        ro = b // 8
        ox[ro, pl.ds(c, 16)] = plsc.load_gather(vxl, jv)
        oy[ro, pl.ds(c, 16)] = plsc.load_gather(vyl, jv)
        oz[ro, pl.ds(c, 16)] = plsc.load_gather(vzl, jv)
        onx[ro, pl.ds(c, 16)] = plsc.load_gather(nxl, jv)
        ony[ro, pl.ds(c, 16)] = plsc.load_gather(nyl, jv)
        onz[ro, pl.ds(c, 16)] = plsc.load_gather(nzl, jv)
        return 0
    lax.fori_loop(0, _VPT // 16, vbody, 0)

    def hbody(b, _):
        s = wid * _HPT + b * 16
        hh = hcpl[pl.ds(s, 16)]
        hrc = _rc(hh)
        jh = _rc(plsc.load_gather(idxl, hrc))
        xhx = plsc.load_gather(vxl, hrc)
        xhy = plsc.load_gather(vyl, hrc)
        xhz = plsc.load_gather(vzl, hrc)
        xjx = plsc.load_gather(vxl, jh)
        xjy = plsc.load_gather(vyl, jh)
        xjz = plsc.load_gather(vzl, jh)
        njx = plsc.load_gather(nxl, jh)
        njy = plsc.load_gather(nyl, jh)
        njz = plsc.load_gather(nzl, jh)
        md = plsc.load_gather(mdl, hrc)
        ext = (njx * (xjx - xhx) + njy * (xjy - xhy) + njz * (xjz - xhz)) > 0.0
        isds = lax.bitwise_and(hh, 7) == 0
        hvi = jnp.where(jnp.logical_and(isds, jnp.logical_not(ext)),
                        jnp.ones((16,), _f32), jnp.zeros((16,), _f32))
        o = b * 16
        ohv[pl.ds(o, 16)] = hvi
        ovh[pl.ds(o, 16)] = md
        return 0
    lax.fori_loop(0, _HPT // 16, hbody, 0)

    ho = [pltpu.async_copy(s, t, sem_out)
          for s, t in ((ox, gxj_h.at[pl.ds(wid * 2, 2)]),
                       (oy, gyj_h.at[pl.ds(wid * 2, 2)]),
                       (oz, gzj_h.at[pl.ds(wid * 2, 2)]),
                       (onx, gnx_h.at[pl.ds(wid * 2, 2)]),
                       (ony, gny_h.at[pl.ds(wid * 2, 2)]),
                       (onz, gnz_h.at[pl.ds(wid * 2, 2)]),
                       (ohv, hvi_h.at[pl.ds(wid * _HPT, _HPT)]),
                       (ovh, vh_h.at[pl.ds(wid * _HPT, _HPT)]))]
    for h in ho:
        h.wait()


def _gather_call(idx_t, md_t, vx, vy, vz, nx, ny, nz, hcp):
    mesh = plsc.VectorSubcoreMesh(core_axis_name="c", subcore_axis_name="s",
                                  num_cores=2)
    out_type = [jax.ShapeDtypeStruct((NR, 128), _f32) for _ in range(6)] + \
               [jax.ShapeDtypeStruct((HCPP,), _f32),
                jax.ShapeDtypeStruct((HCPP,), _f32)]
    t2f = pltpu.VMEM((NR, 128), _f32)
    scratch = [
        pltpu.VMEM((NR, 128), _i32), t2f,
        t2f, t2f, t2f,
        t2f, t2f, t2f,
        pltpu.VMEM((HCPP,), _i32),
        pltpu.VMEM((2, 128), _f32), pltpu.VMEM((2, 128), _f32),
        pltpu.VMEM((2, 128), _f32), pltpu.VMEM((2, 128), _f32),
        pltpu.VMEM((2, 128), _f32), pltpu.VMEM((2, 128), _f32),
        pltpu.VMEM((_HPT,), _f32), pltpu.VMEM((_HPT,), _f32),
        pltpu.SemaphoreType.DMA, pltpu.SemaphoreType.DMA,
    ]
    return pl.kernel(_gather_body, out_type=out_type, mesh=mesh,
                     scratch_types=scratch,
                     compiler_params=pltpu.CompilerParams(
                         needs_layout_passes=False),
                     )(idx_t, md_t, vx, vy, vz, nx, ny, nz, hcp)


def _finish_body(md_ref, gc_ref, gxj_ref, gyj_ref, gzj_ref,
                 gnx_ref, gny_ref, gnz_ref, vnx_ref, vny_ref, vnz_ref,
                 xx_ref, xy_ref, xz_ref, ivx_ref, ivy_ref, ivz_ref,
                 hvi_ref, vh_ref, hw_ref, bp_ref, ip_ref, lhp_ref, rhp_ref,
                 out_ref):
    pos = (lax.broadcasted_iota(_i32, (NR, 128), 0) * 128
           + lax.broadcasted_iota(_i32, (NR, 128), 1))
    valid = pos < N
    v2v = jnp.sqrt(md_ref[...] + 1e-12)
    xx = xx_ref[...]
    xy = xy_ref[...]
    xz = xz_ref[...]
    gnx = gnx_ref[...]
    gny = gny_ref[...]
    gnz = gnz_ref[...]
    vnx = vnx_ref[...]
    vny = vny_ref[...]
    vnz = vnz_ref[...]
    ext = (gnx * (gxj_ref[...] - xx) + gny * (gyj_ref[...] - xy)
           + gnz * (gzj_ref[...] - xz)) > 0.0
    isds = jnp.logical_and((pos % 8) == 0, valid)
    inside = jnp.logical_and(isds, jnp.logical_not(ext))

    def mmean(v, m):
        c = jnp.sum(jnp.where(m, 1.0, 0.0))
        s = jnp.sum(jnp.where(m, v, 0.0))
        return jnp.where(c > 0, s / jnp.maximum(c, 1.0), 0.0)

    gc = gc_ref[...]
    wout = 1.0 / (5.0 * gc + 1.0)
    contact = CONTACT_W * mmean(A1 * wout * jnp.tanh(v2v / A2),
                                jnp.logical_and(isds, jnp.logical_not(inside)))
    insidel = INSIDE_W * mmean(B1 * jnp.tanh(v2v / B2), inside)
    ni = jnp.sqrt(vnx * vnx + vny * vny + vnz * vnz)
    nj = jnp.sqrt(gnx * gnx + gny * gny + gnz * gnz)
    ng = (vnx * gnx + vny * gny + vnz * gnz) / ((ni + 1e-8) * (nj + 1e-8))
    angle = ANGLE_W * mmean(1.0 + ng, jnp.logical_and(v2v < 0.01, valid))

    odel = jnp.sqrt((ivx_ref[...] - xx) ** 2 + (ivy_ref[...] - xy) ** 2
                    + (ivz_ref[...] - xz) ** 2 + 1e-12)
    outside = OUTSIDE_W * jnp.sum(jnp.where(valid, odel * (2.0 * gc) ** 2, 0.0))

    hpos = (lax.broadcasted_iota(_i32, (16, 128), 0) * 128
            + lax.broadcasted_iota(_i32, (16, 128), 1))
    lmask = hpos < HA
    rmask = jnp.logical_and(hpos >= HA, hpos < 2 * HA)
    hv = hvi_ref[...] > 0.5
    nhv = jnp.logical_not(hv)
    v2vh = jnp.sqrt(vh_ref[...] + 1e-12)
    w = -0.1 * hw_ref[...] + 1.0
    vout = w * (C1 * jnp.tanh(v2vh / C2))
    vin = D1 * jnp.tanh(v2vh / D2)
    hco = mmean(vout, jnp.logical_and(lmask, nhv)) + \
          mmean(vout, jnp.logical_and(rmask, nhv))
    hci = mmean(vin, jnp.logical_and(lmask, hv)) + \
          mmean(vin, jnp.logical_and(rmask, hv))
    hand = HCP_W * (hci + hco)

    bp = bp_ref[...] - ip_ref[...]
    pose = POSE_W * jnp.sum(bp * bp)
    hpp = HPP_W * (jnp.sum(lhp_ref[...] ** 2) + jnp.sum(rhp_ref[...] ** 2))
    total = contact + insidel + outside + angle + pose + hpp + hand
    out_ref[...] = total[None, None]


def _finish_call(*args):
    return pl.pallas_call(
        _finish_body,
        out_shape=jax.ShapeDtypeStruct((1, 1), _f32),
    )(*args)


@jax.jit
def kernel(vertices, body_pose, left_hand_pose, right_hand_pose, init_verts,
           init_pose, geodist, hand_contact_prior_weights, faces, ds,
           hand_contact_prior, init_verts_in_contact):
    x = vertices[0]
    vx = jnp.pad(x[:, 0], (0, NP - N))
    vy = jnp.pad(x[:, 1], (0, NP - N))
    vz = jnp.pad(x[:, 2], (0, NP - N))
    f0 = jnp.pad(faces[:, 0], (0, FP - F), constant_values=SINK)
    f1 = jnp.pad(faces[:, 1], (0, FP - F), constant_values=SINK)
    f2 = jnp.pad(faces[:, 2], (0, FP - F), constant_values=SINK)
    ivc = jnp.pad(init_verts_in_contact, (0, NCP - 400), constant_values=SINK)
    hcp = jnp.pad(hand_contact_prior, (0, HCPP - 2 * HA))

    cm2 = _cmask_call(ivc)
    vnx2, vny2, vnz2 = _normals_call(vx, vy, vz, f0, f1, f2)

    a = jnp.pad(x, ((0, NP - N), (0, 5)))
    b = jnp.pad(x.T, ((0, 5), (0, 0)))
    mind2_3, idx_3, gc_3 = _stream_call(a, b, geodist, cm2)
    pad10 = lambda v: jnp.pad(v.reshape(_GRID * _RBR, 128),
                              ((0, NR - _GRID * _RBR), (0, 0)))
    mind2 = pad10(mind2_3)
    idx = pad10(idx_3)
    gc = pad10(gc_3)

    r2 = lambda v: v.reshape(NR, 128)
    gxj, gyj, gzj, gnx, gny, gnz, hvi, vh = _gather_call(
        idx, mind2, r2(vx), r2(vy), r2(vz),
        vnx2, vny2, vnz2, hcp)

    ivx = jnp.pad(init_verts[0, :, 0], (0, NP - N))
    ivy = jnp.pad(init_verts[0, :, 1], (0, NP - N))
    ivz = jnp.pad(init_verts[0, :, 2], (0, NP - N))
    hw = jnp.pad(hand_contact_prior_weights, (0, HCPP - 2 * HA)).reshape(16, 128)
    bp = jnp.pad(body_pose[0], (0, 128 - 69))[None, :]
    ip = jnp.pad(init_pose[0], (0, 128 - 69))[None, :]
    lhp = jnp.pad(left_hand_pose[0], (0, 128 - 45))[None, :]
    rhp = jnp.pad(right_hand_pose[0], (0, 128 - 45))[None, :]

    out = _finish_call(mind2, gc, gxj, gyj, gzj,
                       gnx, gny, gnz, vnx2, vny2, vnz2,
                       r2(vx), r2(vy), r2(vz), r2(ivx), r2(ivy), r2(ivz),
                       hvi.reshape(16, 128), vh.reshape(16, 128), hw,
                       bp, ip, lhp, rhp)
    return out[0, 0]


# speedup vs baseline: 1.0897x; 1.0053x over previous
import functools

import jax
import jax.numpy as jnp
from jax import lax
from jax.experimental import pallas as pl
from jax.experimental.pallas import tpu as pltpu
from jax.experimental.pallas import tpu_sc as plsc

N = 6890
NP = 8192
NR = 64
F = 13776
FP = 13824
FPT = 864
HA = 778
HCPP = 2048
NC0 = 400
SINK = 6900

INSIDE_W = 0.5
OUTSIDE_W = 0.005
CONTACT_W = 10.0
HCP_W = 1.0
POSE_W = 0.01
HPP_W = 0.01
ANGLE_W = 0.1
A1, A2 = 0.04, 0.04
B1, B2 = 0.07, 0.06
C1, C2 = 0.01, 0.01
D1, D2 = 0.023, 0.02
GEO_THRES = 0.3

_f32 = jnp.float32
_i32 = jnp.int32


def _zero2d(ref, nrows):
    def body(i, _):
        r = i // 8
        c = (i % 8) * 16
        ref[r, pl.ds(c, 16)] = jnp.zeros((16,), _f32)
        return 0
    lax.fori_loop(0, nrows * 8, body, 0)


def _cmask_body(ivc_ref, cm_ref):
    iv = ivc_ref[...]
    jj = lax.broadcasted_iota(_i32, (NC0, N), 1)
    hit = jnp.any(iv == jj, axis=0)
    cm_ref[...] = jnp.where(hit, 1.0, 0.0)[None, :]


def _cmask_call(ivc):
    return pl.pallas_call(
        _cmask_body,
        out_shape=jax.ShapeDtypeStruct((1, N), _f32),
    )(ivc.reshape(NC0, 1))


def _normals_body(vx_h, vy_h, vz_h, f0_h, f1_h, f2_h,
                  vnx_h, vny_h, vnz_h,
                  vxl, vyl, vzl, f0l, f1l, f2l,
                  accx, accy, accz, rbuf,
                  svnx, svny, svnz):
    tid = lax.axis_index("s")
    pltpu.sync_copy(vx_h, vxl)
    pltpu.sync_copy(vy_h, vyl)
    pltpu.sync_copy(vz_h, vzl)
    pltpu.sync_copy(f0_h.at[pl.ds(tid * FPT, FPT)], f0l)
    pltpu.sync_copy(f1_h.at[pl.ds(tid * FPT, FPT)], f1l)
    pltpu.sync_copy(f2_h.at[pl.ds(tid * FPT, FPT)], f2l)

    _zero2d(accx, NR)
    _zero2d(accy, NR)
    _zero2d(accz, NR)

    def fbody(b, _):
        s = b * 16
        i0 = f0l[pl.ds(s, 16)]
        i1 = f1l[pl.ds(s, 16)]
        i2 = f2l[pl.ds(s, 16)]
        v0x = plsc.load_gather(vxl, [i0])
        v0y = plsc.load_gather(vyl, [i0])
        v0z = plsc.load_gather(vzl, [i0])
        v1x = plsc.load_gather(vxl, [i1])
        v1y = plsc.load_gather(vyl, [i1])
        v1z = plsc.load_gather(vzl, [i1])
        v2x = plsc.load_gather(vxl, [i2])
        v2y = plsc.load_gather(vyl, [i2])
        v2z = plsc.load_gather(vzl, [i2])
        e1x = v1x - v0x
        e1y = v1y - v0y
        e1z = v1z - v0z
        e2x = v2x - v0x
        e2y = v2y - v0y
        e2z = v2z - v0z
        fnx = e1y * e2z - e1z * e2y
        fny = e1z * e2x - e1x * e2z
        fnz = e1x * e2y - e1y * e2x
        for ii in (i0, i1, i2):
            r = lax.shift_right_logical(ii, 7)
            c = lax.bitwise_and(ii, 127)
            plsc.addupdate_scatter(accx, [r, c], fnx)
            plsc.addupdate_scatter(accy, [r, c], fny)
            plsc.addupdate_scatter(accz, [r, c], fnz)
        return 0
    lax.fori_loop(0, FPT // 16, fbody, 0)

    pltpu.sync_copy(accx, svnx.at[tid])
    pltpu.sync_copy(accy, svny.at[tid])
    pltpu.sync_copy(accz, svnz.at[tid])
    plsc.subcore_barrier()

    rpt = NR // 16
    for src, dst in ((svnx, vnx_h), (svny, vny_h), (svnz, vnz_h)):
        for s in range(16):
            pltpu.sync_copy(src.at[s, pl.ds(tid * rpt, rpt)], rbuf.at[s])

        def rbody(i, _):
            r = i // 8
            c = (i % 8) * 16
            tot = rbuf[0, r, pl.ds(c, 16)]
            for s in range(1, 16):
                tot = tot + rbuf[s, r, pl.ds(c, 16)]
            accx[r, pl.ds(c, 16)] = tot
            return 0
        lax.fori_loop(0, rpt * 8, rbody, 0)
        pltpu.sync_copy(accx.at[pl.ds(0, rpt)], dst.at[pl.ds(tid * rpt, rpt)])


def _normals_call(vx, vy, vz, f0, f1, f2):
    mesh = plsc.VectorSubcoreMesh(core_axis_name="c", subcore_axis_name="s",
                                  num_cores=1)
    out_type = [jax.ShapeDtypeStruct((NR, 128), _f32) for _ in range(3)]
    scratch = [
        pltpu.VMEM((NP,), _f32), pltpu.VMEM((NP,), _f32), pltpu.VMEM((NP,), _f32),
        pltpu.VMEM((FPT,), _i32), pltpu.VMEM((FPT,), _i32), pltpu.VMEM((FPT,), _i32),
        pltpu.VMEM((NR, 128), _f32), pltpu.VMEM((NR, 128), _f32),
        pltpu.VMEM((NR, 128), _f32),
        pltpu.VMEM((16, NR // 16, 128), _f32),
        pltpu.VMEM_SHARED((16, NR, 128), _f32),
        pltpu.VMEM_SHARED((16, NR, 128), _f32),
        pltpu.VMEM_SHARED((16, NR, 128), _f32),
    ]
    return pl.kernel(_normals_body, out_type=out_type, mesh=mesh,
                     scratch_types=scratch,
                     compiler_params=pltpu.CompilerParams(
                         needs_layout_passes=False),
                     )(vx, vy, vz, f0, f1, f2)


RB = 256
_GRID = 27


_RBR = RB // 128


def _stream_body(a_ref, b_ref, g_ref, cm_ref, min_ref, idx_ref, gc_ref):
    a = a_ref[...]
    b = b_ref[...]
    p = lax.dot_general(a, b, (((1,), (0,)), ((), ())),
                        preferred_element_type=_f32)
    aa = a * a
    sqr = (aa[:, 0:1] + aa[:, 1:2]) + aa[:, 2:3]
    bb = b * b
    sqc = (bb[0:1, :] + bb[1:2, :]) + bb[2:3, :]
    d2 = sqr + sqc - 2.0 * p
    d2 = jnp.maximum(d2, 0.0)
    g = g_ref[...]
    d2m = jnp.where(g < GEO_THRES, 1e10, d2)
    m = jnp.min(d2m, axis=1)
    iota = lax.broadcasted_iota(_i32, d2m.shape, 1)
    idx = jnp.min(jnp.where(d2m == m[:, None], iota, N), axis=1)
    gc = jnp.min(jnp.where(cm_ref[...] > 0.5, g, 1e10), axis=1)
    min_ref[...] = m.reshape(1, _RBR, 128)
    idx_ref[...] = idx.reshape(1, _RBR, 128)
    gc_ref[...] = gc.reshape(1, _RBR, 128)


def _stream_call(a, b, geodist, cm):
    return pl.pallas_call(
        _stream_body,
        grid=(_GRID,),
        in_specs=[
            pl.BlockSpec((RB, 8), lambda i: (i, 0)),
            pl.BlockSpec((8, N), lambda i: (0, 0)),
            pl.BlockSpec((RB, N), lambda i: (i, 0)),
            pl.BlockSpec((1, N), lambda i: (0, 0)),
        ],
        out_specs=[
            pl.BlockSpec((1, _RBR, 128), lambda i: (i, 0, 0)),
            pl.BlockSpec((1, _RBR, 128), lambda i: (i, 0, 0)),
            pl.BlockSpec((1, _RBR, 128), lambda i: (i, 0, 0)),
        ],
        out_shape=[
            jax.ShapeDtypeStruct((_GRID, _RBR, 128), _f32),
            jax.ShapeDtypeStruct((_GRID, _RBR, 128), _i32),
            jax.ShapeDtypeStruct((_GRID, _RBR, 128), _f32),
        ],
    )(a, b, geodist, cm)


_VPT = NP // 32
_HPT = HCPP // 32


def _gather_body(idx_h, md_h, vx_h, vy_h, vz_h, nx_h, ny_h, nz_h, hcp_h,
                 gxj_h, gyj_h, gzj_h, gnx_h, gny_h, gnz_h, hvi_h, vh_h,
                 idxl, mdl, vxl, vyl, vzl, nxl, nyl, nzl, hcpl,
                 ox, oy, oz, onx, ony, onz, ohv, ovh, sem_in, sem_out):
    nc = 2
    wid = lax.axis_index("s") * nc + lax.axis_index("c")
    hs = [pltpu.async_copy(s, t, sem_in)
          for s, t in ((idx_h, idxl), (md_h, mdl), (vx_h, vxl), (vy_h, vyl),
                       (vz_h, vzl), (nx_h, nxl), (ny_h, nyl), (nz_h, nzl),
                       (hcp_h, hcpl))]
    for h in hs:
        h.wait()

    def _rc(ii):
        ii = lax.bitwise_and(ii, NP - 1)
        return [lax.shift_right_logical(ii, 7), lax.bitwise_and(ii, 127)]

    def vbody(b, _):
        r = wid * 2 + b // 8
        c = (b % 8) * 16
        jv = _rc(idxl[r, pl.ds(c, 16)])
        ro = b // 8
        ox[ro, pl.ds(c, 16)] = plsc.load_gather(vxl, jv)
        oy[ro, pl.ds(c, 16)] = plsc.load_gather(vyl, jv)
        oz[ro, pl.ds(c, 16)] = plsc.load_gather(vzl, jv)
        onx[ro, pl.ds(c, 16)] = plsc.load_gather(nxl, jv)
        ony[ro, pl.ds(c, 16)] = plsc.load_gather(nyl, jv)
        onz[ro, pl.ds(c, 16)] = plsc.load_gather(nzl, jv)
        return 0
    lax.fori_loop(0, _VPT // 16, vbody, 0)

    def hbody(b, _):
        s = wid * _HPT + b * 16
        hh = hcpl[pl.ds(s, 16)]
        hrc = _rc(hh)
        jh = _rc(plsc.load_gather(idxl, hrc))
        xhx = plsc.load_gather(vxl, hrc)
        xhy = plsc.load_gather(vyl, hrc)
        xhz = plsc.load_gather(vzl, hrc)
        xjx = plsc.load_gather(vxl, jh)
        xjy = plsc.load_gather(vyl, jh)
        xjz = plsc.load_gather(vzl, jh)
        njx = plsc.load_gather(nxl, jh)
        njy = plsc.load_gather(nyl, jh)
        njz = plsc.load_gather(nzl, jh)
        md = plsc.load_gather(mdl, hrc)
        ext = (njx * (xjx - xhx) + njy * (xjy - xhy) + njz * (xjz - xhz)) > 0.0
        isds = lax.bitwise_and(hh, 7) == 0
        hvi = jnp.where(jnp.logical_and(isds, jnp.logical_not(ext)),
                        jnp.ones((16,), _f32), jnp.zeros((16,), _f32))
        o = b * 16
        ohv[pl.ds(o, 16)] = hvi
        ovh[pl.ds(o, 16)] = md
        return 0
    lax.fori_loop(0, _HPT // 16, hbody, 0)

    ho = [pltpu.async_copy(s, t, sem_out)
          for s, t in ((ox, gxj_h.at[pl.ds(wid * 2, 2)]),
                       (oy, gyj_h.at[pl.ds(wid * 2, 2)]),
                       (oz, gzj_h.at[pl.ds(wid * 2, 2)]),
                       (onx, gnx_h.at[pl.ds(wid * 2, 2)]),
                       (ony, gny_h.at[pl.ds(wid * 2, 2)]),
                       (onz, gnz_h.at[pl.ds(wid * 2, 2)]),
                       (ohv, hvi_h.at[wid // 2, pl.ds((wid % 2) * _HPT, _HPT)]),
                       (ovh, vh_h.at[wid // 2, pl.ds((wid % 2) * _HPT, _HPT)]))]
    for h in ho:
        h.wait()


def _gather_call(idx_t, md_t, vx, vy, vz, nx, ny, nz, hcp):
    mesh = plsc.VectorSubcoreMesh(core_axis_name="c", subcore_axis_name="s",
                                  num_cores=2)
    out_type = [jax.ShapeDtypeStruct((NR, 128), _f32) for _ in range(6)] + \
               [jax.ShapeDtypeStruct((16, 128), _f32),
                jax.ShapeDtypeStruct((16, 128), _f32)]
    t2f = pltpu.VMEM((NR, 128), _f32)
    scratch = [
        pltpu.VMEM((NR, 128), _i32), t2f,
        t2f, t2f, t2f,
        t2f, t2f, t2f,
        pltpu.VMEM((HCPP,), _i32),
        pltpu.VMEM((2, 128), _f32), pltpu.VMEM((2, 128), _f32),
        pltpu.VMEM((2, 128), _f32), pltpu.VMEM((2, 128), _f32),
        pltpu.VMEM((2, 128), _f32), pltpu.VMEM((2, 128), _f32),
        pltpu.VMEM((_HPT,), _f32), pltpu.VMEM((_HPT,), _f32),
        pltpu.SemaphoreType.DMA, pltpu.SemaphoreType.DMA,
    ]
    return pl.kernel(_gather_body, out_type=out_type, mesh=mesh,
                     scratch_types=scratch,
                     compiler_params=pltpu.CompilerParams(
                         needs_layout_passes=False),
                     )(idx_t, md_t, vx, vy, vz, nx, ny, nz, hcp)


def _finish_body(md_ref, gc_ref, gxj_ref, gyj_ref, gzj_ref,
                 gnx_ref, gny_ref, gnz_ref, vnx_ref, vny_ref, vnz_ref,
                 xx_ref, xy_ref, xz_ref, ivx_ref, ivy_ref, ivz_ref,
                 hvi_ref, vh_ref, hw_ref, bp_ref, ip_ref, lhp_ref, rhp_ref,
                 out_ref):
    pos = (lax.broadcasted_iota(_i32, (NR, 128), 0) * 128
           + lax.broadcasted_iota(_i32, (NR, 128), 1))
    valid = pos < N
    v2v = jnp.sqrt(md_ref[...] + 1e-12)
    xx = xx_ref[...]
    xy = xy_ref[...]
    xz = xz_ref[...]
    gnx = gnx_ref[...]
    gny = gny_ref[...]
    gnz = gnz_ref[...]
    vnx = vnx_ref[...]
    vny = vny_ref[...]
    vnz = vnz_ref[...]
    ext = (gnx * (gxj_ref[...] - xx) + gny * (gyj_ref[...] - xy)
           + gnz * (gzj_ref[...] - xz)) > 0.0
    isds = jnp.logical_and((pos % 8) == 0, valid)
    inside = jnp.logical_and(isds, jnp.logical_not(ext))

    def mmean(v, m):
        c = jnp.sum(jnp.where(m, 1.0, 0.0))
        s = jnp.sum(jnp.where(m, v, 0.0))
        return jnp.where(c > 0, s / jnp.maximum(c, 1.0), 0.0)

    gc = gc_ref[...]
    wout = 1.0 / (5.0 * gc + 1.0)
    contact = CONTACT_W * mmean(A1 * wout * jnp.tanh(v2v / A2),
                                jnp.logical_and(isds, jnp.logical_not(inside)))
    insidel = INSIDE_W * mmean(B1 * jnp.tanh(v2v / B2), inside)
    ni = jnp.sqrt(vnx * vnx + vny * vny + vnz * vnz)
    nj = jnp.sqrt(gnx * gnx + gny * gny + gnz * gnz)
    ng = (vnx * gnx + vny * gny + vnz * gnz) / ((ni + 1e-8) * (nj + 1e-8))
    angle = ANGLE_W * mmean(1.0 + ng, jnp.logical_and(v2v < 0.01, valid))

    odel = jnp.sqrt((ivx_ref[...] - xx) ** 2 + (ivy_ref[...] - xy) ** 2
                    + (ivz_ref[...] - xz) ** 2 + 1e-12)
    outside = OUTSIDE_W * jnp.sum(jnp.where(valid, odel * (2.0 * gc) ** 2, 0.0))

    hpos = (lax.broadcasted_iota(_i32, (16, 128), 0) * 128
            + lax.broadcasted_iota(_i32, (16, 128), 1))
    lmask = hpos < HA
    rmask = jnp.logical_and(hpos >= HA, hpos < 2 * HA)
    hv = hvi_ref[...] > 0.5
    nhv = jnp.logical_not(hv)
    v2vh = jnp.sqrt(vh_ref[...] + 1e-12)
    w = -0.1 * hw_ref[...] + 1.0
    vout = w * (C1 * jnp.tanh(v2vh / C2))
    vin = D1 * jnp.tanh(v2vh / D2)
    hco = mmean(vout, jnp.logical_and(lmask, nhv)) + \
          mmean(vout, jnp.logical_and(rmask, nhv))
    hci = mmean(vin, jnp.logical_and(lmask, hv)) + \
          mmean(vin, jnp.logical_and(rmask, hv))
    hand = HCP_W * (hci + hco)

    bp = bp_ref[...] - ip_ref[...]
    pose = POSE_W * jnp.sum(bp * bp)
    hpp = HPP_W * (jnp.sum(lhp_ref[...] ** 2) + jnp.sum(rhp_ref[...] ** 2))
    total = contact + insidel + outside + angle + pose + hpp + hand
    out_ref[...] = total[None, None]


def _finish_call(*args):
    return pl.pallas_call(
        _finish_body,
        out_shape=jax.ShapeDtypeStruct((1, 1), _f32),
    )(*args)


@jax.jit
def kernel(vertices, body_pose, left_hand_pose, right_hand_pose, init_verts,
           init_pose, geodist, hand_contact_prior_weights, faces, ds,
           hand_contact_prior, init_verts_in_contact):
    x = vertices[0]
    vx = jnp.pad(x[:, 0], (0, NP - N))
    vy = jnp.pad(x[:, 1], (0, NP - N))
    vz = jnp.pad(x[:, 2], (0, NP - N))
    f0 = jnp.pad(faces[:, 0], (0, FP - F), constant_values=SINK)
    f1 = jnp.pad(faces[:, 1], (0, FP - F), constant_values=SINK)
    f2 = jnp.pad(faces[:, 2], (0, FP - F), constant_values=SINK)
    hcp = jnp.pad(hand_contact_prior, (0, HCPP - 2 * HA))

    cm2 = _cmask_call(init_verts_in_contact)
    vnx2, vny2, vnz2 = _normals_call(vx, vy, vz, f0, f1, f2)

    a = jnp.pad(x, ((0, NP - N), (0, 5)))
    b = jnp.pad(x.T, ((0, 5), (0, 0)))
    mind2_3, idx_3, gc_3 = _stream_call(a, b, geodist, cm2)
    pad10 = lambda v: jnp.pad(v.reshape(_GRID * _RBR, 128),
                              ((0, NR - _GRID * _RBR), (0, 0)))
    mind2 = pad10(mind2_3)
    idx = pad10(idx_3)
    gc = pad10(gc_3)

    r2 = lambda v: v.reshape(NR, 128)
    gxj, gyj, gzj, gnx, gny, gnz, hvi, vh = _gather_call(
        idx, mind2, r2(vx), r2(vy), r2(vz),
        vnx2, vny2, vnz2, hcp)

    ivx = jnp.pad(init_verts[0, :, 0], (0, NP - N))
    ivy = jnp.pad(init_verts[0, :, 1], (0, NP - N))
    ivz = jnp.pad(init_verts[0, :, 2], (0, NP - N))
    hw = jnp.pad(hand_contact_prior_weights, (0, HCPP - 2 * HA)).reshape(16, 128)

    out = _finish_call(mind2, gc, gxj, gyj, gzj,
                       gnx, gny, gnz, vnx2, vny2, vnz2,
                       r2(vx), r2(vy), r2(vz), r2(ivx), r2(ivy), r2(ivz),
                       hvi, vh, hw,
                       body_pose, init_pose, left_hand_pose, right_hand_pose)
    return out[0, 0]
